# Initial kernel scaffold; baseline (speedup 1.0000x reference)
import jax
import jax.numpy as jnp
from jax.experimental import pallas as pl


def kernel(x, edge_index, edge_weight, W1, b1, W2, b2, W3, b3, M1, c1, M2, c2, M3, c3):
    raise NotImplementedError("write your pallas kernel here")


# speedup vs baseline: 2.8215x; 2.8215x over previous
import jax
import jax.numpy as jnp
from jax import lax
from jax.experimental import pallas as pl
from jax.experimental.pallas import tpu as pltpu
from jax.experimental.pallas import tpu_sc as plsc

N = 10000
E = 160000
D = 256

N_PAD = 10240
E_PAD = 163840
NC = 2
NS = 16
NW = NC * NS
EPW = E_PAD // NW
EPS = E_PAD // NS
CH = 128
CJ = EPS // CH
RPS = N_PAD // NS

_f32 = jnp.float32
_i32 = jnp.int32


def _mesh():
    return plsc.VectorSubcoreMesh(core_axis_name="c", subcore_axis_name="s")


def _sc_deg_body(col_hbm, ew_hbm, degp_hbm, colv, ewv, bins):
    c = lax.axis_index("c")
    s = lax.axis_index("s")
    wid = s * NC + c
    pltpu.sync_copy(col_hbm.at[wid], colv)
    pltpu.sync_copy(ew_hbm.at[wid], ewv)

    def zero_body(i, _):
        bins[pl.ds(i * 16, 16)] = jnp.zeros((16,), _f32)
        return 0

    lax.fori_loop(0, N_PAD // 16, zero_body, 0)

    def add_body(i, _):
        ci = colv[pl.ds(i * 16, 16)]
        wi = ewv[pl.ds(i * 16, 16)]
        plsc.addupdate_scatter(bins, [ci], wi)
        return 0

    lax.fori_loop(0, EPW // 16, add_body, 0)
    pltpu.sync_copy(bins, degp_hbm.at[wid])


def _sc_deg(col2, ew2):
    fn = pl.kernel(
        _sc_deg_body,
        out_type=jax.ShapeDtypeStruct((NW, N_PAD), _f32),
        mesh=_mesh(),
        compiler_params=pltpu.CompilerParams(needs_layout_passes=False, use_tc_tiling_on_sc=False),
        scratch_types=[
            pltpu.VMEM((EPW,), _i32),
            pltpu.VMEM((EPW,), _f32),
            pltpu.VMEM((N_PAD,), _f32),
        ],
    )
    return fn(col2, ew2)


def _agg_loop(m_hbm, init_hbm, out_hbm, rowv, colv, normv, gbuf, acc, sem):
    c = lax.axis_index("c")
    s = lax.axis_index("s")
    for p in range(2):
        q = c * 2 + p
        pltpu.sync_copy(init_hbm.at[q, pl.ds(s * RPS, RPS)], acc.at[pl.ds(s * RPS, RPS)])
        plsc.subcore_barrier()

        def chunk_body(j, _):
            cp = pltpu.async_copy(m_hbm.at[q].at[rowv.at[j]], gbuf, sem)
            cp.wait()

            def edge_body(e, _):
                ns = plsc.load_gather(
                    normv, [jnp.full((16,), j, _i32), jnp.full((16,), e, _i32)]
                )
                for k in range(4):
                    sl = pl.ds(k * 16, 16)
                    gbuf[e, sl] = gbuf[e, sl] * ns
                return 0

            lax.fori_loop(0, CH, edge_body, 0)
            pltpu.sync_copy(gbuf, acc.at[colv.at[j]], add=True)
            return 0

        lax.fori_loop(0, CJ, chunk_body, 0)
        plsc.subcore_barrier()
        pltpu.sync_copy(acc.at[pl.ds(s * RPS, RPS)], out_hbm.at[q, pl.ds(s * RPS, RPS)])
        plsc.subcore_barrier()


def _sc_agg1_body(m_hbm, init_hbm, row_hbm, col_hbm, ew_hbm, dinv_hbm,
                  out_hbm, norm_hbm,
                  rowv, colv, ewv, normv, dinvv, gbuf, acc, sem):
    c = lax.axis_index("c")
    s = lax.axis_index("s")
    pltpu.sync_copy(row_hbm.at[s], rowv)
    pltpu.sync_copy(col_hbm.at[s], colv)
    pltpu.sync_copy(ew_hbm.at[s], ewv)
    pltpu.sync_copy(dinv_hbm, dinvv)

    def norm_body(j, _):
        for k in range(8):
            sl = pl.ds(k * 16, 16)
            r16 = rowv[j, sl]
            c16 = colv[j, sl]
            w16 = ewv[j, sl]
            normv[j, sl] = (
                plsc.load_gather(dinvv, [r16]) * w16 * plsc.load_gather(dinvv, [c16])
            )
        return 0

    lax.fori_loop(0, CJ, norm_body, 0)

    @pl.when(c == 0)
    def _():
        pltpu.sync_copy(normv, norm_hbm.at[s])

    _agg_loop(m_hbm, init_hbm, out_hbm, rowv, colv, normv, gbuf, acc, sem)


def _sc_agg2_body(m_hbm, init_hbm, row_hbm, col_hbm, norm_hbm, out_hbm,
                  rowv, colv, normv, gbuf, acc, sem):
    s = lax.axis_index("s")
    pltpu.sync_copy(row_hbm.at[s], rowv)
    pltpu.sync_copy(col_hbm.at[s], colv)
    pltpu.sync_copy(norm_hbm.at[s], normv)
    _agg_loop(m_hbm, init_hbm, out_hbm, rowv, colv, normv, gbuf, acc, sem)


def _sc_agg1(m, init, row3, col3, ew3, dinv):
    fn = pl.kernel(
        _sc_agg1_body,
        out_type=(
            jax.ShapeDtypeStruct((4, N_PAD, 64), _f32),
            jax.ShapeDtypeStruct((NS, CJ, CH), _f32),
        ),
        mesh=_mesh(),
        compiler_params=pltpu.CompilerParams(needs_layout_passes=False, use_tc_tiling_on_sc=False),
        scratch_types=[
            pltpu.VMEM((CJ, CH), _i32),
            pltpu.VMEM((CJ, CH), _i32),
            pltpu.VMEM((CJ, CH), _f32),
            pltpu.VMEM((CJ, CH), _f32),
            pltpu.VMEM((N_PAD,), _f32),
            pltpu.VMEM((CH, 64), _f32),
            pltpu.VMEM_SHARED((N_PAD, 64), _f32),
            pltpu.SemaphoreType.DMA,
        ],
    )
    return fn(m, init, row3, col3, ew3, dinv)


def _sc_agg2(m, init, row3, col3, norm3):
    fn = pl.kernel(
        _sc_agg2_body,
        out_type=jax.ShapeDtypeStruct((4, N_PAD, 64), _f32),
        mesh=_mesh(),
        compiler_params=pltpu.CompilerParams(needs_layout_passes=False, use_tc_tiling_on_sc=False),
        scratch_types=[
            pltpu.VMEM((CJ, CH), _i32),
            pltpu.VMEM((CJ, CH), _i32),
            pltpu.VMEM((CJ, CH), _f32),
            pltpu.VMEM((CH, 64), _f32),
            pltpu.VMEM_SHARED((N_PAD, 64), _f32),
            pltpu.SemaphoreType.DMA,
        ],
    )
    return fn(m, init, row3, col3, norm3)


def _sc_edge_body(a_hbm, b_hbm, row_hbm, col_hbm, ew_hbm, m1w_hbm, z1_hbm,
                  rowv, colv, ewv, m1wv, abuf, bbuf, zbuf, sema, semb):
    c = lax.axis_index("c")
    s = lax.axis_index("s")
    pltpu.sync_copy(row_hbm.at[s], rowv)
    pltpu.sync_copy(col_hbm.at[s], colv)
    pltpu.sync_copy(ew_hbm.at[s], ewv)
    pltpu.sync_copy(m1w_hbm.at[c], m1wv)
    w16 = [m1wv[pl.ds(k * 16, 16)] for k in range(8)]

    def chunk_body(j, _):
        ca = pltpu.async_copy(a_hbm.at[c].at[rowv.at[j]], abuf, sema)
        cb = pltpu.async_copy(b_hbm.at[c].at[colv.at[j]], bbuf, semb)
        ca.wait()
        cb.wait()

        def edge_body(e, _):
            es = plsc.load_gather(
                ewv, [jnp.full((16,), j, _i32), jnp.full((16,), e, _i32)]
            )
            for k in range(8):
                sl = pl.ds(k * 16, 16)
                zbuf[e, sl] = jnp.maximum(
                    abuf[e, sl] + bbuf[e, sl] + es * w16[k], 0.0
                )
            return 0

        lax.fori_loop(0, CH, edge_body, 0)
        pltpu.sync_copy(zbuf, z1_hbm.at[c, pl.ds(s * EPS + j * CH, CH)])
        return 0

    lax.fori_loop(0, CJ, chunk_body, 0)


def _sc_edge(a, b, row3, col3, ew3, m1w):
    fn = pl.kernel(
        _sc_edge_body,
        out_type=jax.ShapeDtypeStruct((NC, E_PAD, 128), _f32),
        mesh=_mesh(),
        compiler_params=pltpu.CompilerParams(needs_layout_passes=False, use_tc_tiling_on_sc=False),
        scratch_types=[
            pltpu.VMEM((CJ, CH), _i32),
            pltpu.VMEM((CJ, CH), _i32),
            pltpu.VMEM((CJ, CH), _f32),
            pltpu.VMEM((128,), _f32),
            pltpu.VMEM((CH, 128), _f32),
            pltpu.VMEM((CH, 128), _f32),
            pltpu.VMEM((CH, 128), _f32),
            pltpu.SemaphoreType.DMA,
            pltpu.SemaphoreType.DMA,
        ],
    )
    return fn(a, b, row3, col3, ew3, m1w)


BLKN = 1024
BLKE = 2048


def _split2(ref, val):
    ref[0] = val[:, :128]
    ref[1] = val[:, 128:]


def _split4(ref, val):
    for q in range(4):
        ref[q] = val[:, q * 64:(q + 1) * 64]


def _tc_prep_body(degp_ref, x_ref, w_ref, dinv_ref, m_ref, init_ref):
    deg = jnp.sum(degp_ref[...], axis=0) + 1.0
    dinv = lax.rsqrt(deg)
    dinv_ref[...] = dinv
    m = jnp.dot(x_ref[...], w_ref[...], preferred_element_type=_f32)
    _split4(m_ref, m)
    _split4(init_ref, m * (dinv * dinv)[:, None])


def _tc_prep(degp, x_p, w1):
    return pl.pallas_call(
        _tc_prep_body,
        grid=(N_PAD // BLKN,),
        in_specs=[
            pl.BlockSpec((NW, BLKN), lambda i: (0, i)),
            pl.BlockSpec((BLKN, D), lambda i: (i, 0)),
            pl.BlockSpec((D, D), lambda i: (0, 0)),
        ],
        out_specs=[
            pl.BlockSpec((BLKN,), lambda i: (i,)),
            pl.BlockSpec((4, BLKN, 64), lambda i: (0, i, 0)),
            pl.BlockSpec((4, BLKN, 64), lambda i: (0, i, 0)),
        ],
        out_shape=[
            jax.ShapeDtypeStruct((N_PAD,), _f32),
            jax.ShapeDtypeStruct((4, N_PAD, 64), _f32),
            jax.ShapeDtypeStruct((4, N_PAD, 64), _f32),
        ],
    )(degp, x_p, w1)


def _tc_layer_body(agg_ref, b_ref, w_ref, dinv_ref, m_ref, init_ref):
    h = jnp.concatenate([agg_ref[q] for q in range(4)], axis=-1) + b_ref[...]
    h = jnp.maximum(h, 0.0)
    dinv = dinv_ref[...]
    m = jnp.dot(h, w_ref[...], preferred_element_type=_f32)
    _split4(m_ref, m)
    _split4(init_ref, m * (dinv * dinv)[:, None])


def _tc_layer(agg, b_row, w, dinv):
    return pl.pallas_call(
        _tc_layer_body,
        grid=(N_PAD // BLKN,),
        in_specs=[
            pl.BlockSpec((4, BLKN, 64), lambda i: (0, i, 0)),
            pl.BlockSpec((1, D), lambda i: (0, 0)),
            pl.BlockSpec((D, D), lambda i: (0, 0)),
            pl.BlockSpec((BLKN,), lambda i: (i,)),
        ],
        out_specs=[
            pl.BlockSpec((4, BLKN, 64), lambda i: (0, i, 0)),
            pl.BlockSpec((4, BLKN, 64), lambda i: (0, i, 0)),
        ],
        out_shape=[
            jax.ShapeDtypeStruct((4, N_PAD, 64), _f32),
            jax.ShapeDtypeStruct((4, N_PAD, 64), _f32),
        ],
    )(agg, b_row, w, dinv)


def _tc_ab_body(agg_ref, b3_ref, m1a_ref, m1b_ref, c1_ref, a_ref, bo_ref):
    h = jnp.concatenate([agg_ref[q] for q in range(4)], axis=-1) + b3_ref[...]
    a = jnp.dot(h, m1a_ref[...], preferred_element_type=_f32) + c1_ref[...]
    bb = jnp.dot(h, m1b_ref[...], preferred_element_type=_f32)
    _split2(a_ref, a)
    _split2(bo_ref, bb)


def _tc_ab(agg, b3_row, m1a, m1b, c1_row):
    return pl.pallas_call(
        _tc_ab_body,
        grid=(N_PAD // BLKN,),
        in_specs=[
            pl.BlockSpec((4, BLKN, 64), lambda i: (0, i, 0)),
            pl.BlockSpec((1, D), lambda i: (0, 0)),
            pl.BlockSpec((D, D), lambda i: (0, 0)),
            pl.BlockSpec((D, D), lambda i: (0, 0)),
            pl.BlockSpec((1, D), lambda i: (0, 0)),
        ],
        out_specs=[
            pl.BlockSpec((NC, BLKN, 128), lambda i: (0, i, 0)),
            pl.BlockSpec((NC, BLKN, 128), lambda i: (0, i, 0)),
        ],
        out_shape=[
            jax.ShapeDtypeStruct((NC, N_PAD, 128), _f32),
            jax.ShapeDtypeStruct((NC, N_PAD, 128), _f32),
        ],
    )(agg, b3_row, m1a, m1b, c1_row)


def _tc_mlp_body(z1_ref, m2_ref, c2_ref, m3_ref, c3_ref, out_ref):
    z = jnp.concatenate([z1_ref[0], z1_ref[1]], axis=-1)
    z2 = jnp.maximum(
        jnp.dot(z, m2_ref[...], preferred_element_type=_f32) + c2_ref[...], 0.0
    )
    t = jnp.sum(z2 * m3_ref[...], axis=1) + c3_ref[0, 0]
    out_ref[...] = t.reshape(1, 1, BLKE)


def _tc_mlp(z1, m2, c2_row, m3_row, c3s):
    return pl.pallas_call(
        _tc_mlp_body,
        grid=(E_PAD // BLKE,),
        in_specs=[
            pl.BlockSpec((NC, BLKE, 128), lambda i: (0, i, 0)),
            pl.BlockSpec((D, D), lambda i: (0, 0)),
            pl.BlockSpec((1, D), lambda i: (0, 0)),
            pl.BlockSpec((1, D), lambda i: (0, 0)),
            pl.BlockSpec(memory_space=pltpu.SMEM),
        ],
        out_specs=pl.BlockSpec((1, 1, BLKE), lambda i: (i, 0, 0)),
        out_shape=jax.ShapeDtypeStruct((E_PAD // BLKE, 1, BLKE), _f32),
    )(z1, m2, c2_row, m3_row, c3s)


def kernel(x, edge_index, edge_weight, W1, b1, W2, b2, W3, b3,
           M1, c1, M2, c2, M3, c3):
    row = edge_index[0]
    col = edge_index[1]
    x_p = jnp.pad(x, ((0, N_PAD - N), (0, 0)))
    row_p = jnp.pad(row, (0, E_PAD - E))
    col_p = jnp.pad(col, (0, E_PAD - E))
    ew_p = jnp.pad(edge_weight, (0, E_PAD - E))

    col2 = col_p.reshape(NW, EPW)
    ew2 = ew_p.reshape(NW, EPW)
    row3 = row_p.reshape(NS, CJ, CH)
    col3 = col_p.reshape(NS, CJ, CH)
    ew3 = ew_p.reshape(NS, CJ, CH)

    m1a = M1[:D]
    m1b = M1[D:2 * D]
    m1w = M1[2 * D].reshape(NC, 128)

    b1r = b1.reshape(1, D)
    b2r = b2.reshape(1, D)
    b3r = b3.reshape(1, D)
    c1r = c1.reshape(1, D)
    c2r = c2.reshape(1, D)
    c3s = c3.reshape(1, 1)
    m3r = M3.reshape(1, D)

    degp = _sc_deg(col2, ew2)
    dinv, m1s, init1 = _tc_prep(degp, x_p, W1)

    agg1, norm3 = _sc_agg1(m1s, init1, row3, col3, ew3, dinv)
    m2s, init2 = _tc_layer(agg1, b1r, W2, dinv)
    agg2 = _sc_agg2(m2s, init2, row3, col3, norm3)
    m3s, init3 = _tc_layer(agg2, b2r, W3, dinv)
    agg3 = _sc_agg2(m3s, init3, row3, col3, norm3)

    a_nodes, b_nodes = _tc_ab(agg3, b3r, m1a, m1b, c1r)
    z1 = _sc_edge(a_nodes, b_nodes, row3, col3, ew3, m1w)

    out = _tc_mlp(z1, M2, c2r, m3r, c3s)
    return out.reshape(E_PAD)[:E]


# speedup vs baseline: 2.8486x; 1.0096x over previous
import jax
import jax.numpy as jnp
from jax import lax
from jax.experimental import pallas as pl
from jax.experimental.pallas import tpu as pltpu
from jax.experimental.pallas import tpu_sc as plsc

N = 10000
E = 160000
D = 256

N_PAD = 10240
E_PAD = 163840
NC = 2
NS = 16
NW = NC * NS
EPW = E_PAD // NW
EPS = E_PAD // NS
CH = 128
CJ = EPS // CH
RPS = N_PAD // NS

_f32 = jnp.float32
_i32 = jnp.int32


def _mesh():
    return plsc.VectorSubcoreMesh(core_axis_name="c", subcore_axis_name="s")


def _sc_deg_body(col_hbm, ew_hbm, degp_hbm, colv, ewv, bins):
    c = lax.axis_index("c")
    s = lax.axis_index("s")
    wid = s * NC + c
    pltpu.sync_copy(col_hbm.at[wid], colv)
    pltpu.sync_copy(ew_hbm.at[wid], ewv)

    def zero_body(i, _):
        bins[pl.ds(i * 16, 16)] = jnp.zeros((16,), _f32)
        return 0

    lax.fori_loop(0, N_PAD // 16, zero_body, 0)

    def add_body(i, _):
        ci = colv[pl.ds(i * 16, 16)]
        wi = ewv[pl.ds(i * 16, 16)]
        plsc.addupdate_scatter(bins, [ci], wi)
        return 0

    lax.fori_loop(0, EPW // 16, add_body, 0)
    pltpu.sync_copy(bins, degp_hbm.at[wid])


def _sc_deg(col2, ew2):
    fn = pl.kernel(
        _sc_deg_body,
        out_type=jax.ShapeDtypeStruct((NW, N_PAD), _f32),
        mesh=_mesh(),
        compiler_params=pltpu.CompilerParams(needs_layout_passes=False, use_tc_tiling_on_sc=False),
        scratch_types=[
            pltpu.VMEM((EPW,), _i32),
            pltpu.VMEM((EPW,), _f32),
            pltpu.VMEM((N_PAD,), _f32),
        ],
    )
    return fn(col2, ew2)


def _agg_loop(m_hbm, init_hbm, out_hbm, rowv, colv, normv, gbuf, acc, sem):
    c = lax.axis_index("c")
    s = lax.axis_index("s")
    for p in range(2):
        q = c * 2 + p
        pltpu.sync_copy(init_hbm.at[q, pl.ds(s * RPS, RPS)], acc.at[pl.ds(s * RPS, RPS)])
        plsc.subcore_barrier()

        def chunk_body(j, _):
            cp = pltpu.async_copy(m_hbm.at[q].at[rowv.at[j]], gbuf, sem)
            cp.wait()

            def edge_body(e, _):
                ns = plsc.load_gather(
                    normv, [jnp.full((16,), j, _i32), jnp.full((16,), e, _i32)]
                )
                for k in range(4):
                    sl = pl.ds(k * 16, 16)
                    gbuf[e, sl] = gbuf[e, sl] * ns
                return 0

            lax.fori_loop(0, CH, edge_body, 0, unroll=4)
            pltpu.sync_copy(gbuf, acc.at[colv.at[j]], add=True)
            return 0

        lax.fori_loop(0, CJ, chunk_body, 0)
        plsc.subcore_barrier()
        pltpu.sync_copy(acc.at[pl.ds(s * RPS, RPS)], out_hbm.at[q, pl.ds(s * RPS, RPS)])
        plsc.subcore_barrier()


def _sc_agg1_body(m_hbm, init_hbm, row_hbm, col_hbm, ew_hbm, dinv_hbm,
                  out_hbm, norm_hbm,
                  rowv, colv, ewv, normv, dinvv, gbuf, acc, sem):
    c = lax.axis_index("c")
    s = lax.axis_index("s")
    pltpu.sync_copy(row_hbm.at[s], rowv)
    pltpu.sync_copy(col_hbm.at[s], colv)
    pltpu.sync_copy(ew_hbm.at[s], ewv)
    pltpu.sync_copy(dinv_hbm, dinvv)

    def norm_body(j, _):
        for k in range(8):
            sl = pl.ds(k * 16, 16)
            r16 = rowv[j, sl]
            c16 = colv[j, sl]
            w16 = ewv[j, sl]
            normv[j, sl] = (
                plsc.load_gather(dinvv, [r16]) * w16 * plsc.load_gather(dinvv, [c16])
            )
        return 0

    lax.fori_loop(0, CJ, norm_body, 0)

    @pl.when(c == 0)
    def _():
        pltpu.sync_copy(normv, norm_hbm.at[s])

    _agg_loop(m_hbm, init_hbm, out_hbm, rowv, colv, normv, gbuf, acc, sem)


def _sc_agg2_body(m_hbm, init_hbm, row_hbm, col_hbm, norm_hbm, out_hbm,
                  rowv, colv, normv, gbuf, acc, sem):
    s = lax.axis_index("s")
    pltpu.sync_copy(row_hbm.at[s], rowv)
    pltpu.sync_copy(col_hbm.at[s], colv)
    pltpu.sync_copy(norm_hbm.at[s], normv)
    _agg_loop(m_hbm, init_hbm, out_hbm, rowv, colv, normv, gbuf, acc, sem)


def _sc_agg1(m, init, row3, col3, ew3, dinv):
    fn = pl.kernel(
        _sc_agg1_body,
        out_type=(
            jax.ShapeDtypeStruct((4, N_PAD, 64), _f32),
            jax.ShapeDtypeStruct((NS, CJ, CH), _f32),
        ),
        mesh=_mesh(),
        compiler_params=pltpu.CompilerParams(needs_layout_passes=False, use_tc_tiling_on_sc=False),
        scratch_types=[
            pltpu.VMEM((CJ, CH), _i32),
            pltpu.VMEM((CJ, CH), _i32),
            pltpu.VMEM((CJ, CH), _f32),
            pltpu.VMEM((CJ, CH), _f32),
            pltpu.VMEM((N_PAD,), _f32),
            pltpu.VMEM((CH, 64), _f32),
            pltpu.VMEM_SHARED((N_PAD, 64), _f32),
            pltpu.SemaphoreType.DMA,
        ],
    )
    return fn(m, init, row3, col3, ew3, dinv)


def _sc_agg2(m, init, row3, col3, norm3):
    fn = pl.kernel(
        _sc_agg2_body,
        out_type=jax.ShapeDtypeStruct((4, N_PAD, 64), _f32),
        mesh=_mesh(),
        compiler_params=pltpu.CompilerParams(needs_layout_passes=False, use_tc_tiling_on_sc=False),
        scratch_types=[
            pltpu.VMEM((CJ, CH), _i32),
            pltpu.VMEM((CJ, CH), _i32),
            pltpu.VMEM((CJ, CH), _f32),
            pltpu.VMEM((CH, 64), _f32),
            pltpu.VMEM_SHARED((N_PAD, 64), _f32),
            pltpu.SemaphoreType.DMA,
        ],
    )
    return fn(m, init, row3, col3, norm3)


def _sc_edge_body(a_hbm, b_hbm, row_hbm, col_hbm, ew_hbm, m1w_hbm, z1_hbm,
                  rowv, colv, ewv, m1wv, abuf, bbuf, zbuf, sema, semb):
    c = lax.axis_index("c")
    s = lax.axis_index("s")
    pltpu.sync_copy(row_hbm.at[s], rowv)
    pltpu.sync_copy(col_hbm.at[s], colv)
    pltpu.sync_copy(ew_hbm.at[s], ewv)
    pltpu.sync_copy(m1w_hbm.at[c], m1wv)
    w16 = [m1wv[pl.ds(k * 16, 16)] for k in range(8)]

    def chunk_body(j, _):
        ca = pltpu.async_copy(a_hbm.at[c].at[rowv.at[j]], abuf, sema)
        cb = pltpu.async_copy(b_hbm.at[c].at[colv.at[j]], bbuf, semb)
        ca.wait()
        cb.wait()

        def edge_body(e, _):
            es = plsc.load_gather(
                ewv, [jnp.full((16,), j, _i32), jnp.full((16,), e, _i32)]
            )
            for k in range(8):
                sl = pl.ds(k * 16, 16)
                zbuf[e, sl] = jnp.maximum(
                    abuf[e, sl] + bbuf[e, sl] + es * w16[k], 0.0
                )
            return 0

        lax.fori_loop(0, CH, edge_body, 0, unroll=4)
        pltpu.sync_copy(zbuf, z1_hbm.at[c, pl.ds(s * EPS + j * CH, CH)])
        return 0

    lax.fori_loop(0, CJ, chunk_body, 0)


def _sc_edge(a, b, row3, col3, ew3, m1w):
    fn = pl.kernel(
        _sc_edge_body,
        out_type=jax.ShapeDtypeStruct((NC, E_PAD, 128), _f32),
        mesh=_mesh(),
        compiler_params=pltpu.CompilerParams(needs_layout_passes=False, use_tc_tiling_on_sc=False),
        scratch_types=[
            pltpu.VMEM((CJ, CH), _i32),
            pltpu.VMEM((CJ, CH), _i32),
            pltpu.VMEM((CJ, CH), _f32),
            pltpu.VMEM((128,), _f32),
            pltpu.VMEM((CH, 128), _f32),
            pltpu.VMEM((CH, 128), _f32),
            pltpu.VMEM((CH, 128), _f32),
            pltpu.SemaphoreType.DMA,
            pltpu.SemaphoreType.DMA,
        ],
    )
    return fn(a, b, row3, col3, ew3, m1w)


BLKN = 1024
BLKE = 2048


def _split2(ref, val):
    ref[0] = val[:, :128]
    ref[1] = val[:, 128:]


def _split4(ref, val):
    for q in range(4):
        ref[q] = val[:, q * 64:(q + 1) * 64]


def _tc_prep_body(degp_ref, x_ref, w_ref, dinv_ref, m_ref, init_ref):
    deg = jnp.sum(degp_ref[...], axis=0) + 1.0
    dinv = lax.rsqrt(deg)
    dinv_ref[...] = dinv
    m = jnp.dot(x_ref[...], w_ref[...], preferred_element_type=_f32)
    _split4(m_ref, m)
    _split4(init_ref, m * (dinv * dinv)[:, None])


def _tc_prep(degp, x_p, w1):
    return pl.pallas_call(
        _tc_prep_body,
        grid=(N_PAD // BLKN,),
        in_specs=[
            pl.BlockSpec((NW, BLKN), lambda i: (0, i)),
            pl.BlockSpec((BLKN, D), lambda i: (i, 0)),
            pl.BlockSpec((D, D), lambda i: (0, 0)),
        ],
        out_specs=[
            pl.BlockSpec((BLKN,), lambda i: (i,)),
            pl.BlockSpec((4, BLKN, 64), lambda i: (0, i, 0)),
            pl.BlockSpec((4, BLKN, 64), lambda i: (0, i, 0)),
        ],
        out_shape=[
            jax.ShapeDtypeStruct((N_PAD,), _f32),
            jax.ShapeDtypeStruct((4, N_PAD, 64), _f32),
            jax.ShapeDtypeStruct((4, N_PAD, 64), _f32),
        ],
    )(degp, x_p, w1)


def _tc_layer_body(agg_ref, b_ref, w_ref, dinv_ref, m_ref, init_ref):
    h = jnp.concatenate([agg_ref[q] for q in range(4)], axis=-1) + b_ref[...]
    h = jnp.maximum(h, 0.0)
    dinv = dinv_ref[...]
    m = jnp.dot(h, w_ref[...], preferred_element_type=_f32)
    _split4(m_ref, m)
    _split4(init_ref, m * (dinv * dinv)[:, None])


def _tc_layer(agg, b_row, w, dinv):
    return pl.pallas_call(
        _tc_layer_body,
        grid=(N_PAD // BLKN,),
        in_specs=[
            pl.BlockSpec((4, BLKN, 64), lambda i: (0, i, 0)),
            pl.BlockSpec((1, D), lambda i: (0, 0)),
            pl.BlockSpec((D, D), lambda i: (0, 0)),
            pl.BlockSpec((BLKN,), lambda i: (i,)),
        ],
        out_specs=[
            pl.BlockSpec((4, BLKN, 64), lambda i: (0, i, 0)),
            pl.BlockSpec((4, BLKN, 64), lambda i: (0, i, 0)),
        ],
        out_shape=[
            jax.ShapeDtypeStruct((4, N_PAD, 64), _f32),
            jax.ShapeDtypeStruct((4, N_PAD, 64), _f32),
        ],
    )(agg, b_row, w, dinv)


def _tc_ab_body(agg_ref, b3_ref, m1a_ref, m1b_ref, c1_ref, a_ref, bo_ref):
    h = jnp.concatenate([agg_ref[q] for q in range(4)], axis=-1) + b3_ref[...]
    a = jnp.dot(h, m1a_ref[...], preferred_element_type=_f32) + c1_ref[...]
    bb = jnp.dot(h, m1b_ref[...], preferred_element_type=_f32)
    _split2(a_ref, a)
    _split2(bo_ref, bb)


def _tc_ab(agg, b3_row, m1a, m1b, c1_row):
    return pl.pallas_call(
        _tc_ab_body,
        grid=(N_PAD // BLKN,),
        in_specs=[
            pl.BlockSpec((4, BLKN, 64), lambda i: (0, i, 0)),
            pl.BlockSpec((1, D), lambda i: (0, 0)),
            pl.BlockSpec((D, D), lambda i: (0, 0)),
            pl.BlockSpec((D, D), lambda i: (0, 0)),
            pl.BlockSpec((1, D), lambda i: (0, 0)),
        ],
        out_specs=[
            pl.BlockSpec((NC, BLKN, 128), lambda i: (0, i, 0)),
            pl.BlockSpec((NC, BLKN, 128), lambda i: (0, i, 0)),
        ],
        out_shape=[
            jax.ShapeDtypeStruct((NC, N_PAD, 128), _f32),
            jax.ShapeDtypeStruct((NC, N_PAD, 128), _f32),
        ],
    )(agg, b3_row, m1a, m1b, c1_row)


def _tc_mlp_body(z1_ref, m2_ref, c2_ref, m3_ref, c3_ref, out_ref):
    z = jnp.concatenate([z1_ref[0], z1_ref[1]], axis=-1)
    z2 = jnp.maximum(
        jnp.dot(z, m2_ref[...], preferred_element_type=_f32) + c2_ref[...], 0.0
    )
    t = jnp.sum(z2 * m3_ref[...], axis=1) + c3_ref[0, 0]
    out_ref[...] = t.reshape(1, 1, BLKE)


def _tc_mlp(z1, m2, c2_row, m3_row, c3s):
    return pl.pallas_call(
        _tc_mlp_body,
        grid=(E_PAD // BLKE,),
        in_specs=[
            pl.BlockSpec((NC, BLKE, 128), lambda i: (0, i, 0)),
            pl.BlockSpec((D, D), lambda i: (0, 0)),
            pl.BlockSpec((1, D), lambda i: (0, 0)),
            pl.BlockSpec((1, D), lambda i: (0, 0)),
            pl.BlockSpec(memory_space=pltpu.SMEM),
        ],
        out_specs=pl.BlockSpec((1, 1, BLKE), lambda i: (i, 0, 0)),
        out_shape=jax.ShapeDtypeStruct((E_PAD // BLKE, 1, BLKE), _f32),
    )(z1, m2, c2_row, m3_row, c3s)


def kernel(x, edge_index, edge_weight, W1, b1, W2, b2, W3, b3,
           M1, c1, M2, c2, M3, c3):
    row = edge_index[0]
    col = edge_index[1]
    x_p = jnp.pad(x, ((0, N_PAD - N), (0, 0)))
    row_p = jnp.pad(row, (0, E_PAD - E))
    col_p = jnp.pad(col, (0, E_PAD - E))
    ew_p = jnp.pad(edge_weight, (0, E_PAD - E))

    col2 = col_p.reshape(NW, EPW)
    ew2 = ew_p.reshape(NW, EPW)
    row3 = row_p.reshape(NS, CJ, CH)
    col3 = col_p.reshape(NS, CJ, CH)
    ew3 = ew_p.reshape(NS, CJ, CH)

    m1a = M1[:D]
    m1b = M1[D:2 * D]
    m1w = M1[2 * D].reshape(NC, 128)

    b1r = b1.reshape(1, D)
    b2r = b2.reshape(1, D)
    b3r = b3.reshape(1, D)
    c1r = c1.reshape(1, D)
    c2r = c2.reshape(1, D)
    c3s = c3.reshape(1, 1)
    m3r = M3.reshape(1, D)

    degp = _sc_deg(col2, ew2)
    dinv, m1s, init1 = _tc_prep(degp, x_p, W1)

    agg1, norm3 = _sc_agg1(m1s, init1, row3, col3, ew3, dinv)
    m2s, init2 = _tc_layer(agg1, b1r, W2, dinv)
    agg2 = _sc_agg2(m2s, init2, row3, col3, norm3)
    m3s, init3 = _tc_layer(agg2, b2r, W3, dinv)
    agg3 = _sc_agg2(m3s, init3, row3, col3, norm3)

    a_nodes, b_nodes = _tc_ab(agg3, b3r, m1a, m1b, c1r)
    z1 = _sc_edge(a_nodes, b_nodes, row3, col3, ew3, m1w)

    out = _tc_mlp(z1, M2, c2r, m3r, c3s)
    return out.reshape(E_PAD)[:E]


# speedup vs baseline: 3.6929x; 1.2964x over previous
import jax
import jax.numpy as jnp
from jax import lax
from jax.experimental import pallas as pl
from jax.experimental.pallas import tpu as pltpu
from jax.experimental.pallas import tpu_sc as plsc

N = 10000
E = 160000
D = 256

N_PAD = 10240
E_PAD = 163840
NC = 2
NS = 16
NW = NC * NS
EPW = E_PAD // NW
EPS = E_PAD // NS
CH = 128
CJ = EPS // CH
RPS = N_PAD // NS

_f32 = jnp.float32
_i32 = jnp.int32


def _mesh():
    return plsc.VectorSubcoreMesh(core_axis_name="c", subcore_axis_name="s")


def _sc_deg_body(col_hbm, ew_hbm, degp_hbm, colv, ewv, bins):
    c = lax.axis_index("c")
    s = lax.axis_index("s")
    wid = s * NC + c
    pltpu.sync_copy(col_hbm.at[wid], colv)
    pltpu.sync_copy(ew_hbm.at[wid], ewv)

    def zero_body(i, _):
        bins[pl.ds(i * 16, 16)] = jnp.zeros((16,), _f32)
        return 0

    lax.fori_loop(0, N_PAD // 16, zero_body, 0)

    def add_body(i, _):
        ci = colv[pl.ds(i * 16, 16)]
        wi = ewv[pl.ds(i * 16, 16)]
        plsc.addupdate_scatter(bins, [ci], wi)
        return 0

    lax.fori_loop(0, EPW // 16, add_body, 0)
    pltpu.sync_copy(bins, degp_hbm.at[wid])


def _sc_deg(col2, ew2):
    fn = pl.kernel(
        _sc_deg_body,
        out_type=jax.ShapeDtypeStruct((NW, N_PAD), _f32),
        mesh=_mesh(),
        compiler_params=pltpu.CompilerParams(needs_layout_passes=False, use_tc_tiling_on_sc=False),
        scratch_types=[
            pltpu.VMEM((EPW,), _i32),
            pltpu.VMEM((EPW,), _f32),
            pltpu.VMEM((N_PAD,), _f32),
        ],
    )
    return fn(col2, ew2)


def _agg_loop(m_hbm, init_hbm, out_hbm, rowv, colv, normv, gbuf, acc, sem):
    c = lax.axis_index("c")
    s = lax.axis_index("s")
    for p in range(2):
        q = c * 2 + p
        pltpu.sync_copy(init_hbm.at[q, pl.ds(s * RPS, RPS)], acc.at[pl.ds(s * RPS, RPS)])
        plsc.subcore_barrier()

        def chunk_body(j, _):
            cp = pltpu.async_copy(m_hbm.at[q].at[rowv.at[j]], gbuf, sem)
            cp.wait()

            @plsc.parallel_loop(0, CH, unroll=4)
            def edge_body(e):
                ns = plsc.load_gather(
                    normv, [jnp.full((16,), j, _i32), jnp.full((16,), e, _i32)]
                )
                for k in range(4):
                    sl = pl.ds(k * 16, 16)
                    gbuf[e, sl] = gbuf[e, sl] * ns
            pltpu.sync_copy(gbuf, acc.at[colv.at[j]], add=True)
            return 0

        lax.fori_loop(0, CJ, chunk_body, 0)
        plsc.subcore_barrier()
        pltpu.sync_copy(acc.at[pl.ds(s * RPS, RPS)], out_hbm.at[q, pl.ds(s * RPS, RPS)])
        plsc.subcore_barrier()


def _sc_agg1_body(m_hbm, init_hbm, row_hbm, col_hbm, ew_hbm, dinv_hbm,
                  out_hbm, norm_hbm,
                  rowv, colv, ewv, normv, dinvv, gbuf, acc, sem):
    c = lax.axis_index("c")
    s = lax.axis_index("s")
    pltpu.sync_copy(row_hbm.at[s], rowv)
    pltpu.sync_copy(col_hbm.at[s], colv)
    pltpu.sync_copy(ew_hbm.at[s], ewv)
    pltpu.sync_copy(dinv_hbm, dinvv)

    def norm_body(j, _):
        for k in range(8):
            sl = pl.ds(k * 16, 16)
            r16 = rowv[j, sl]
            c16 = colv[j, sl]
            w16 = ewv[j, sl]
            normv[j, sl] = (
                plsc.load_gather(dinvv, [r16]) * w16 * plsc.load_gather(dinvv, [c16])
            )
        return 0

    lax.fori_loop(0, CJ, norm_body, 0)

    @pl.when(c == 0)
    def _():
        pltpu.sync_copy(normv, norm_hbm.at[s])

    _agg_loop(m_hbm, init_hbm, out_hbm, rowv, colv, normv, gbuf, acc, sem)


def _sc_agg2_body(m_hbm, init_hbm, row_hbm, col_hbm, norm_hbm, out_hbm,
                  rowv, colv, normv, gbuf, acc, sem):
    s = lax.axis_index("s")
    pltpu.sync_copy(row_hbm.at[s], rowv)
    pltpu.sync_copy(col_hbm.at[s], colv)
    pltpu.sync_copy(norm_hbm.at[s], normv)
    _agg_loop(m_hbm, init_hbm, out_hbm, rowv, colv, normv, gbuf, acc, sem)


def _sc_agg1(m, init, row3, col3, ew3, dinv):
    fn = pl.kernel(
        _sc_agg1_body,
        out_type=(
            jax.ShapeDtypeStruct((4, N_PAD, 64), _f32),
            jax.ShapeDtypeStruct((NS, CJ, CH), _f32),
        ),
        mesh=_mesh(),
        compiler_params=pltpu.CompilerParams(needs_layout_passes=False, use_tc_tiling_on_sc=False),
        scratch_types=[
            pltpu.VMEM((CJ, CH), _i32),
            pltpu.VMEM((CJ, CH), _i32),
            pltpu.VMEM((CJ, CH), _f32),
            pltpu.VMEM((CJ, CH), _f32),
            pltpu.VMEM((N_PAD,), _f32),
            pltpu.VMEM((CH, 64), _f32),
            pltpu.VMEM_SHARED((N_PAD, 64), _f32),
            pltpu.SemaphoreType.DMA,
        ],
    )
    return fn(m, init, row3, col3, ew3, dinv)


def _sc_agg2(m, init, row3, col3, norm3):
    fn = pl.kernel(
        _sc_agg2_body,
        out_type=jax.ShapeDtypeStruct((4, N_PAD, 64), _f32),
        mesh=_mesh(),
        compiler_params=pltpu.CompilerParams(needs_layout_passes=False, use_tc_tiling_on_sc=False),
        scratch_types=[
            pltpu.VMEM((CJ, CH), _i32),
            pltpu.VMEM((CJ, CH), _i32),
            pltpu.VMEM((CJ, CH), _f32),
            pltpu.VMEM((CH, 64), _f32),
            pltpu.VMEM_SHARED((N_PAD, 64), _f32),
            pltpu.SemaphoreType.DMA,
        ],
    )
    return fn(m, init, row3, col3, norm3)


def _sc_edge_body(a_hbm, b_hbm, row_hbm, col_hbm, ew_hbm, m1w_hbm, z1_hbm,
                  rowv, colv, ewv, m1wv, abuf, bbuf, zbuf, sema, semb):
    c = lax.axis_index("c")
    s = lax.axis_index("s")
    pltpu.sync_copy(row_hbm.at[s], rowv)
    pltpu.sync_copy(col_hbm.at[s], colv)
    pltpu.sync_copy(ew_hbm.at[s], ewv)
    pltpu.sync_copy(m1w_hbm.at[c], m1wv)
    w16 = [m1wv[pl.ds(k * 16, 16)] for k in range(8)]

    def chunk_body(j, _):
        ca = pltpu.async_copy(a_hbm.at[c].at[rowv.at[j]], abuf, sema)
        cb = pltpu.async_copy(b_hbm.at[c].at[colv.at[j]], bbuf, semb)
        ca.wait()
        cb.wait()

        @plsc.parallel_loop(0, CH, unroll=4)
        def edge_body(e):
            es = plsc.load_gather(
                ewv, [jnp.full((16,), j, _i32), jnp.full((16,), e, _i32)]
            )
            for k in range(8):
                sl = pl.ds(k * 16, 16)
                zbuf[e, sl] = jnp.maximum(
                    abuf[e, sl] + bbuf[e, sl] + es * w16[k], 0.0
                )
        pltpu.sync_copy(zbuf, z1_hbm.at[c, pl.ds(s * EPS + j * CH, CH)])
        return 0

    lax.fori_loop(0, CJ, chunk_body, 0)


def _sc_edge(a, b, row3, col3, ew3, m1w):
    fn = pl.kernel(
        _sc_edge_body,
        out_type=jax.ShapeDtypeStruct((NC, E_PAD, 128), _f32),
        mesh=_mesh(),
        compiler_params=pltpu.CompilerParams(needs_layout_passes=False, use_tc_tiling_on_sc=False),
        scratch_types=[
            pltpu.VMEM((CJ, CH), _i32),
            pltpu.VMEM((CJ, CH), _i32),
            pltpu.VMEM((CJ, CH), _f32),
            pltpu.VMEM((128,), _f32),
            pltpu.VMEM((CH, 128), _f32),
            pltpu.VMEM((CH, 128), _f32),
            pltpu.VMEM((CH, 128), _f32),
            pltpu.SemaphoreType.DMA,
            pltpu.SemaphoreType.DMA,
        ],
    )
    return fn(a, b, row3, col3, ew3, m1w)


BLKN = 1024
BLKE = 2048


def _split2(ref, val):
    ref[0] = val[:, :128]
    ref[1] = val[:, 128:]


def _split4(ref, val):
    for q in range(4):
        ref[q] = val[:, q * 64:(q + 1) * 64]


def _tc_prep_body(degp_ref, x_ref, w_ref, dinv_ref, m_ref, init_ref):
    deg = jnp.sum(degp_ref[...], axis=0) + 1.0
    dinv = lax.rsqrt(deg)
    dinv_ref[...] = dinv
    m = jnp.dot(x_ref[...], w_ref[...], preferred_element_type=_f32)
    _split4(m_ref, m)
    _split4(init_ref, m * (dinv * dinv)[:, None])


def _tc_prep(degp, x_p, w1):
    return pl.pallas_call(
        _tc_prep_body,
        grid=(N_PAD // BLKN,),
        in_specs=[
            pl.BlockSpec((NW, BLKN), lambda i: (0, i)),
            pl.BlockSpec((BLKN, D), lambda i: (i, 0)),
            pl.BlockSpec((D, D), lambda i: (0, 0)),
        ],
        out_specs=[
            pl.BlockSpec((BLKN,), lambda i: (i,)),
            pl.BlockSpec((4, BLKN, 64), lambda i: (0, i, 0)),
            pl.BlockSpec((4, BLKN, 64), lambda i: (0, i, 0)),
        ],
        out_shape=[
            jax.ShapeDtypeStruct((N_PAD,), _f32),
            jax.ShapeDtypeStruct((4, N_PAD, 64), _f32),
            jax.ShapeDtypeStruct((4, N_PAD, 64), _f32),
        ],
    )(degp, x_p, w1)


def _tc_layer_body(agg_ref, b_ref, w_ref, dinv_ref, m_ref, init_ref):
    h = jnp.concatenate([agg_ref[q] for q in range(4)], axis=-1) + b_ref[...]
    h = jnp.maximum(h, 0.0)
    dinv = dinv_ref[...]
    m = jnp.dot(h, w_ref[...], preferred_element_type=_f32)
    _split4(m_ref, m)
    _split4(init_ref, m * (dinv * dinv)[:, None])


def _tc_layer(agg, b_row, w, dinv):
    return pl.pallas_call(
        _tc_layer_body,
        grid=(N_PAD // BLKN,),
        in_specs=[
            pl.BlockSpec((4, BLKN, 64), lambda i: (0, i, 0)),
            pl.BlockSpec((1, D), lambda i: (0, 0)),
            pl.BlockSpec((D, D), lambda i: (0, 0)),
            pl.BlockSpec((BLKN,), lambda i: (i,)),
        ],
        out_specs=[
            pl.BlockSpec((4, BLKN, 64), lambda i: (0, i, 0)),
            pl.BlockSpec((4, BLKN, 64), lambda i: (0, i, 0)),
        ],
        out_shape=[
            jax.ShapeDtypeStruct((4, N_PAD, 64), _f32),
            jax.ShapeDtypeStruct((4, N_PAD, 64), _f32),
        ],
    )(agg, b_row, w, dinv)


def _tc_ab_body(agg_ref, b3_ref, m1a_ref, m1b_ref, c1_ref, a_ref, bo_ref):
    h = jnp.concatenate([agg_ref[q] for q in range(4)], axis=-1) + b3_ref[...]
    a = jnp.dot(h, m1a_ref[...], preferred_element_type=_f32) + c1_ref[...]
    bb = jnp.dot(h, m1b_ref[...], preferred_element_type=_f32)
    _split2(a_ref, a)
    _split2(bo_ref, bb)


def _tc_ab(agg, b3_row, m1a, m1b, c1_row):
    return pl.pallas_call(
        _tc_ab_body,
        grid=(N_PAD // BLKN,),
        in_specs=[
            pl.BlockSpec((4, BLKN, 64), lambda i: (0, i, 0)),
            pl.BlockSpec((1, D), lambda i: (0, 0)),
            pl.BlockSpec((D, D), lambda i: (0, 0)),
            pl.BlockSpec((D, D), lambda i: (0, 0)),
            pl.BlockSpec((1, D), lambda i: (0, 0)),
        ],
        out_specs=[
            pl.BlockSpec((NC, BLKN, 128), lambda i: (0, i, 0)),
            pl.BlockSpec((NC, BLKN, 128), lambda i: (0, i, 0)),
        ],
        out_shape=[
            jax.ShapeDtypeStruct((NC, N_PAD, 128), _f32),
            jax.ShapeDtypeStruct((NC, N_PAD, 128), _f32),
        ],
    )(agg, b3_row, m1a, m1b, c1_row)


def _tc_mlp_body(z1_ref, m2_ref, c2_ref, m3_ref, c3_ref, out_ref):
    z = jnp.concatenate([z1_ref[0], z1_ref[1]], axis=-1)
    z2 = jnp.maximum(
        jnp.dot(z, m2_ref[...], preferred_element_type=_f32) + c2_ref[...], 0.0
    )
    t = jnp.sum(z2 * m3_ref[...], axis=1) + c3_ref[0, 0]
    out_ref[...] = t.reshape(1, 1, BLKE)


def _tc_mlp(z1, m2, c2_row, m3_row, c3s):
    return pl.pallas_call(
        _tc_mlp_body,
        grid=(E_PAD // BLKE,),
        in_specs=[
            pl.BlockSpec((NC, BLKE, 128), lambda i: (0, i, 0)),
            pl.BlockSpec((D, D), lambda i: (0, 0)),
            pl.BlockSpec((1, D), lambda i: (0, 0)),
            pl.BlockSpec((1, D), lambda i: (0, 0)),
            pl.BlockSpec(memory_space=pltpu.SMEM),
        ],
        out_specs=pl.BlockSpec((1, 1, BLKE), lambda i: (i, 0, 0)),
        out_shape=jax.ShapeDtypeStruct((E_PAD // BLKE, 1, BLKE), _f32),
    )(z1, m2, c2_row, m3_row, c3s)


def kernel(x, edge_index, edge_weight, W1, b1, W2, b2, W3, b3,
           M1, c1, M2, c2, M3, c3):
    row = edge_index[0]
    col = edge_index[1]
    x_p = jnp.pad(x, ((0, N_PAD - N), (0, 0)))
    row_p = jnp.pad(row, (0, E_PAD - E))
    col_p = jnp.pad(col, (0, E_PAD - E))
    ew_p = jnp.pad(edge_weight, (0, E_PAD - E))

    col2 = col_p.reshape(NW, EPW)
    ew2 = ew_p.reshape(NW, EPW)
    row3 = row_p.reshape(NS, CJ, CH)
    col3 = col_p.reshape(NS, CJ, CH)
    ew3 = ew_p.reshape(NS, CJ, CH)

    m1a = M1[:D]
    m1b = M1[D:2 * D]
    m1w = M1[2 * D].reshape(NC, 128)

    b1r = b1.reshape(1, D)
    b2r = b2.reshape(1, D)
    b3r = b3.reshape(1, D)
    c1r = c1.reshape(1, D)
    c2r = c2.reshape(1, D)
    c3s = c3.reshape(1, 1)
    m3r = M3.reshape(1, D)

    degp = _sc_deg(col2, ew2)
    dinv, m1s, init1 = _tc_prep(degp, x_p, W1)

    agg1, norm3 = _sc_agg1(m1s, init1, row3, col3, ew3, dinv)
    m2s, init2 = _tc_layer(agg1, b1r, W2, dinv)
    agg2 = _sc_agg2(m2s, init2, row3, col3, norm3)
    m3s, init3 = _tc_layer(agg2, b2r, W3, dinv)
    agg3 = _sc_agg2(m3s, init3, row3, col3, norm3)

    a_nodes, b_nodes = _tc_ab(agg3, b3r, m1a, m1b, c1r)
    z1 = _sc_edge(a_nodes, b_nodes, row3, col3, ew3, m1w)

    out = _tc_mlp(z1, M2, c2r, m3r, c3s)
    return out.reshape(E_PAD)[:E]


# speedup vs baseline: 3.7020x; 1.0025x over previous
import jax
import jax.numpy as jnp
from jax import lax
from jax.experimental import pallas as pl
from jax.experimental.pallas import tpu as pltpu
from jax.experimental.pallas import tpu_sc as plsc

N = 10000
E = 160000
D = 256

N_PAD = 10240
E_PAD = 163840
NC = 2
NS = 16
NW = NC * NS
EPW = E_PAD // NW
EPS = E_PAD // NS
CH = 128
CJ = EPS // CH
RPS = N_PAD // NS

_f32 = jnp.float32
_i32 = jnp.int32


def _mesh():
    return plsc.VectorSubcoreMesh(core_axis_name="c", subcore_axis_name="s")


def _sc_deg_body(col_hbm, ew_hbm, degp_hbm, colv, ewv, bins):
    c = lax.axis_index("c")
    s = lax.axis_index("s")
    wid = s * NC + c
    pltpu.sync_copy(col_hbm.at[wid], colv)
    pltpu.sync_copy(ew_hbm.at[wid], ewv)

    def zero_body(i, _):
        bins[pl.ds(i * 16, 16)] = jnp.zeros((16,), _f32)
        return 0

    lax.fori_loop(0, N_PAD // 16, zero_body, 0)

    def add_body(i, _):
        ci = colv[pl.ds(i * 16, 16)]
        wi = ewv[pl.ds(i * 16, 16)]
        plsc.addupdate_scatter(bins, [ci], wi)
        return 0

    lax.fori_loop(0, EPW // 16, add_body, 0)
    pltpu.sync_copy(bins, degp_hbm.at[wid])


def _sc_deg(col2, ew2):
    fn = pl.kernel(
        _sc_deg_body,
        out_type=jax.ShapeDtypeStruct((NW, N_PAD), _f32),
        mesh=_mesh(),
        compiler_params=pltpu.CompilerParams(needs_layout_passes=False, use_tc_tiling_on_sc=False),
        scratch_types=[
            pltpu.VMEM((EPW,), _i32),
            pltpu.VMEM((EPW,), _f32),
            pltpu.VMEM((N_PAD,), _f32),
        ],
    )
    return fn(col2, ew2)


def _agg_loop(m_hbm, init_hbm, out_hbm, rowv, colv, normv, gbuf, acc, sem):
    c = lax.axis_index("c")
    s = lax.axis_index("s")
    for p in range(2):
        q = c * 2 + p
        pltpu.sync_copy(init_hbm.at[q, pl.ds(s * RPS, RPS)], acc.at[pl.ds(s * RPS, RPS)])
        plsc.subcore_barrier()

        def chunk_body(j, _):
            cp = pltpu.async_copy(m_hbm.at[q].at[rowv.at[j]], gbuf, sem)
            cp.wait()

            @plsc.parallel_loop(0, CH, unroll=4)
            def edge_body(e):
                ns = plsc.load_gather(
                    normv, [jnp.full((16,), j, _i32), jnp.full((16,), e, _i32)]
                )
                for k in range(4):
                    sl = pl.ds(k * 16, 16)
                    gbuf[e, sl] = gbuf[e, sl] * ns
            pltpu.sync_copy(gbuf, acc.at[colv.at[j]], add=True)
            return 0

        lax.fori_loop(0, CJ, chunk_body, 0)
        plsc.subcore_barrier()
        pltpu.sync_copy(acc.at[pl.ds(s * RPS, RPS)], out_hbm.at[q, pl.ds(s * RPS, RPS)])
        plsc.subcore_barrier()


def _sc_agg1_body(m_hbm, init_hbm, row_hbm, col_hbm, ew_hbm, dinv_hbm,
                  out_hbm, norm_hbm,
                  rowv, colv, ewv, normv, dinvv, gbuf, acc, sem):
    c = lax.axis_index("c")
    s = lax.axis_index("s")
    pltpu.sync_copy(row_hbm.at[s], rowv)
    pltpu.sync_copy(col_hbm.at[s], colv)
    pltpu.sync_copy(ew_hbm.at[s], ewv)
    pltpu.sync_copy(dinv_hbm, dinvv)

    def norm_body(j, _):
        for k in range(8):
            sl = pl.ds(k * 16, 16)
            r16 = rowv[j, sl]
            c16 = colv[j, sl]
            w16 = ewv[j, sl]
            normv[j, sl] = (
                plsc.load_gather(dinvv, [r16]) * w16 * plsc.load_gather(dinvv, [c16])
            )
        return 0

    lax.fori_loop(0, CJ, norm_body, 0)

    @pl.when(c == 0)
    def _():
        pltpu.sync_copy(normv, norm_hbm.at[s])

    _agg_loop(m_hbm, init_hbm, out_hbm, rowv, colv, normv, gbuf, acc, sem)


def _sc_agg2_body(m_hbm, init_hbm, row_hbm, col_hbm, norm_hbm, out_hbm,
                  rowv, colv, normv, gbuf, acc, sem):
    s = lax.axis_index("s")
    pltpu.sync_copy(row_hbm.at[s], rowv)
    pltpu.sync_copy(col_hbm.at[s], colv)
    pltpu.sync_copy(norm_hbm.at[s], normv)
    _agg_loop(m_hbm, init_hbm, out_hbm, rowv, colv, normv, gbuf, acc, sem)


def _sc_agg1(m, init, row3, col3, ew3, dinv):
    fn = pl.kernel(
        _sc_agg1_body,
        out_type=(
            jax.ShapeDtypeStruct((4, N_PAD, 64), _f32),
            jax.ShapeDtypeStruct((NS, CJ, CH), _f32),
        ),
        mesh=_mesh(),
        compiler_params=pltpu.CompilerParams(needs_layout_passes=False, use_tc_tiling_on_sc=False),
        scratch_types=[
            pltpu.VMEM((CJ, CH), _i32),
            pltpu.VMEM((CJ, CH), _i32),
            pltpu.VMEM((CJ, CH), _f32),
            pltpu.VMEM((CJ, CH), _f32),
            pltpu.VMEM((N_PAD,), _f32),
            pltpu.VMEM((CH, 64), _f32),
            pltpu.VMEM_SHARED((N_PAD, 64), _f32),
            pltpu.SemaphoreType.DMA,
        ],
    )
    return fn(m, init, row3, col3, ew3, dinv)


def _sc_agg2(m, init, row3, col3, norm3):
    fn = pl.kernel(
        _sc_agg2_body,
        out_type=jax.ShapeDtypeStruct((4, N_PAD, 64), _f32),
        mesh=_mesh(),
        compiler_params=pltpu.CompilerParams(needs_layout_passes=False, use_tc_tiling_on_sc=False),
        scratch_types=[
            pltpu.VMEM((CJ, CH), _i32),
            pltpu.VMEM((CJ, CH), _i32),
            pltpu.VMEM((CJ, CH), _f32),
            pltpu.VMEM((CH, 64), _f32),
            pltpu.VMEM_SHARED((N_PAD, 64), _f32),
            pltpu.SemaphoreType.DMA,
        ],
    )
    return fn(m, init, row3, col3, norm3)


def _round_bf16(v):
    u = lax.bitcast_convert_type(v, _i32)
    lsb = lax.shift_right_logical(u, 16) & 1
    u = (u + 0x7FFF + lsb) & jnp.int32(-65536)
    return lax.bitcast_convert_type(u, _f32)


def _sc_edge_body(a_hbm, b_hbm, row_hbm, col_hbm, ew_hbm, m1w_hbm, z1_hbm,
                  rowv, colv, ewv, m1wv, abuf, bbuf, zbuf, sema, semb):
    c = lax.axis_index("c")
    s = lax.axis_index("s")
    pltpu.sync_copy(row_hbm.at[s], rowv)
    pltpu.sync_copy(col_hbm.at[s], colv)
    pltpu.sync_copy(ew_hbm.at[s], ewv)
    pltpu.sync_copy(m1w_hbm.at[c], m1wv)
    @plsc.parallel_loop(0, CJ, unroll=2)
    def _round_ew(i):
        for k in range(CH // 16):
            sl = pl.ds(k * 16, 16)
            ewv[i, sl] = _round_bf16(ewv[i, sl])

    w16 = [_round_bf16(m1wv[pl.ds(k * 16, 16)]) for k in range(8)]

    def chunk_body(j, _):
        ca = pltpu.async_copy(a_hbm.at[c].at[rowv.at[j]], abuf, sema)
        cb = pltpu.async_copy(b_hbm.at[c].at[colv.at[j]], bbuf, semb)
        ca.wait()
        cb.wait()

        @plsc.parallel_loop(0, CH, unroll=4)
        def edge_body(e):
            es = plsc.load_gather(
                ewv, [jnp.full((16,), j, _i32), jnp.full((16,), e, _i32)]
            )
            for k in range(8):
                sl = pl.ds(k * 16, 16)
                zbuf[e, sl] = jnp.maximum(
                    abuf[e, sl] + bbuf[e, sl] + es * w16[k], 0.0
                )
        pltpu.sync_copy(zbuf, z1_hbm.at[c, pl.ds(s * EPS + j * CH, CH)])
        return 0

    lax.fori_loop(0, CJ, chunk_body, 0)


def _sc_edge(a, b, row3, col3, ew3, m1w):
    fn = pl.kernel(
        _sc_edge_body,
        out_type=jax.ShapeDtypeStruct((NC, E_PAD, 128), _f32),
        mesh=_mesh(),
        compiler_params=pltpu.CompilerParams(needs_layout_passes=False, use_tc_tiling_on_sc=False),
        scratch_types=[
            pltpu.VMEM((CJ, CH), _i32),
            pltpu.VMEM((CJ, CH), _i32),
            pltpu.VMEM((CJ, CH), _f32),
            pltpu.VMEM((128,), _f32),
            pltpu.VMEM((CH, 128), _f32),
            pltpu.VMEM((CH, 128), _f32),
            pltpu.VMEM((CH, 128), _f32),
            pltpu.SemaphoreType.DMA,
            pltpu.SemaphoreType.DMA,
        ],
    )
    return fn(a, b, row3, col3, ew3, m1w)


BLKN = 1024
BLKE = 2048


def _split2(ref, val):
    ref[0] = val[:, :128]
    ref[1] = val[:, 128:]


def _split4(ref, val):
    for q in range(4):
        ref[q] = val[:, q * 64:(q + 1) * 64]


def _tc_prep_body(degp_ref, x_ref, w_ref, dinv_ref, m_ref, init_ref):
    deg = jnp.sum(degp_ref[...], axis=0) + 1.0
    dinv = 1.0 / jnp.sqrt(deg)
    dinv_ref[...] = dinv
    m = jnp.dot(x_ref[...], w_ref[...], preferred_element_type=_f32)
    _split4(m_ref, m)
    _split4(init_ref, m * (dinv * dinv)[:, None])


def _tc_prep(degp, x_p, w1):
    return pl.pallas_call(
        _tc_prep_body,
        grid=(N_PAD // BLKN,),
        in_specs=[
            pl.BlockSpec((NW, BLKN), lambda i: (0, i)),
            pl.BlockSpec((BLKN, D), lambda i: (i, 0)),
            pl.BlockSpec((D, D), lambda i: (0, 0)),
        ],
        out_specs=[
            pl.BlockSpec((BLKN,), lambda i: (i,)),
            pl.BlockSpec((4, BLKN, 64), lambda i: (0, i, 0)),
            pl.BlockSpec((4, BLKN, 64), lambda i: (0, i, 0)),
        ],
        out_shape=[
            jax.ShapeDtypeStruct((N_PAD,), _f32),
            jax.ShapeDtypeStruct((4, N_PAD, 64), _f32),
            jax.ShapeDtypeStruct((4, N_PAD, 64), _f32),
        ],
    )(degp, x_p, w1)


def _tc_layer_body(agg_ref, b_ref, w_ref, dinv_ref, m_ref, init_ref):
    h = jnp.concatenate([agg_ref[q] for q in range(4)], axis=-1) + b_ref[...]
    h = jnp.maximum(h, 0.0)
    dinv = dinv_ref[...]
    m = jnp.dot(h, w_ref[...], preferred_element_type=_f32)
    _split4(m_ref, m)
    _split4(init_ref, m * (dinv * dinv)[:, None])


def _tc_layer(agg, b_row, w, dinv):
    return pl.pallas_call(
        _tc_layer_body,
        grid=(N_PAD // BLKN,),
        in_specs=[
            pl.BlockSpec((4, BLKN, 64), lambda i: (0, i, 0)),
            pl.BlockSpec((1, D), lambda i: (0, 0)),
            pl.BlockSpec((D, D), lambda i: (0, 0)),
            pl.BlockSpec((BLKN,), lambda i: (i,)),
        ],
        out_specs=[
            pl.BlockSpec((4, BLKN, 64), lambda i: (0, i, 0)),
            pl.BlockSpec((4, BLKN, 64), lambda i: (0, i, 0)),
        ],
        out_shape=[
            jax.ShapeDtypeStruct((4, N_PAD, 64), _f32),
            jax.ShapeDtypeStruct((4, N_PAD, 64), _f32),
        ],
    )(agg, b_row, w, dinv)


def _tc_ab_body(agg_ref, b3_ref, m1a_ref, m1b_ref, c1_ref, a_ref, bo_ref):
    h = jnp.concatenate([agg_ref[q] for q in range(4)], axis=-1) + b3_ref[...]
    a = jnp.dot(h, m1a_ref[...], preferred_element_type=_f32) + c1_ref[...]
    bb = jnp.dot(h, m1b_ref[...], preferred_element_type=_f32)
    _split2(a_ref, a)
    _split2(bo_ref, bb)


def _tc_ab(agg, b3_row, m1a, m1b, c1_row):
    return pl.pallas_call(
        _tc_ab_body,
        grid=(N_PAD // BLKN,),
        in_specs=[
            pl.BlockSpec((4, BLKN, 64), lambda i: (0, i, 0)),
            pl.BlockSpec((1, D), lambda i: (0, 0)),
            pl.BlockSpec((D, D), lambda i: (0, 0)),
            pl.BlockSpec((D, D), lambda i: (0, 0)),
            pl.BlockSpec((1, D), lambda i: (0, 0)),
        ],
        out_specs=[
            pl.BlockSpec((NC, BLKN, 128), lambda i: (0, i, 0)),
            pl.BlockSpec((NC, BLKN, 128), lambda i: (0, i, 0)),
        ],
        out_shape=[
            jax.ShapeDtypeStruct((NC, N_PAD, 128), _f32),
            jax.ShapeDtypeStruct((NC, N_PAD, 128), _f32),
        ],
    )(agg, b3_row, m1a, m1b, c1_row)


def _tc_mlp_body(z1_ref, m2_ref, c2_ref, m3p_ref, c3_ref, out_ref):
    z = jnp.concatenate([z1_ref[0], z1_ref[1]], axis=-1)
    z2 = jnp.maximum(
        jnp.dot(z, m2_ref[...], preferred_element_type=_f32) + c2_ref[...], 0.0
    )
    t = jnp.dot(z2, m3p_ref[...], preferred_element_type=_f32)
    out_ref[...] = t[:, 0:1] + c3_ref[0, 0]


def _tc_mlp(z1, m2, c2_row, m3p, c3s):
    return pl.pallas_call(
        _tc_mlp_body,
        grid=(E_PAD // BLKE,),
        in_specs=[
            pl.BlockSpec((NC, BLKE, 128), lambda i: (0, i, 0)),
            pl.BlockSpec((D, D), lambda i: (0, 0)),
            pl.BlockSpec((1, D), lambda i: (0, 0)),
            pl.BlockSpec((D, 128), lambda i: (0, 0)),
            pl.BlockSpec(memory_space=pltpu.SMEM),
        ],
        out_specs=pl.BlockSpec((BLKE, 1), lambda i: (i, 0)),
        out_shape=jax.ShapeDtypeStruct((E_PAD, 1), _f32),
    )(z1, m2, c2_row, m3p, c3s)


def kernel(x, edge_index, edge_weight, W1, b1, W2, b2, W3, b3,
           M1, c1, M2, c2, M3, c3):
    row = edge_index[0]
    col = edge_index[1]
    x_p = jnp.pad(x, ((0, N_PAD - N), (0, 0)))
    row_p = jnp.pad(row, (0, E_PAD - E))
    col_p = jnp.pad(col, (0, E_PAD - E))
    ew_p = jnp.pad(edge_weight, (0, E_PAD - E))

    col2 = col_p.reshape(NW, EPW)
    ew2 = ew_p.reshape(NW, EPW)
    row3 = row_p.reshape(NS, CJ, CH)
    col3 = col_p.reshape(NS, CJ, CH)
    ew3 = ew_p.reshape(NS, CJ, CH)

    m1a = M1[:D]
    m1b = M1[D:2 * D]
    m1w = M1[2 * D].reshape(NC, 128)

    b1r = b1.reshape(1, D)
    b2r = b2.reshape(1, D)
    b3r = b3.reshape(1, D)
    c1r = c1.reshape(1, D)
    c2r = c2.reshape(1, D)
    c3s = c3.reshape(1, 1)
    m3p = jnp.pad(M3, ((0, 0), (0, 127)))

    degp = _sc_deg(col2, ew2)
    dinv, m1s, init1 = _tc_prep(degp, x_p, W1)

    agg1, norm3 = _sc_agg1(m1s, init1, row3, col3, ew3, dinv)
    m2s, init2 = _tc_layer(agg1, b1r, W2, dinv)
    agg2 = _sc_agg2(m2s, init2, row3, col3, norm3)
    m3s, init3 = _tc_layer(agg2, b2r, W3, dinv)
    agg3 = _sc_agg2(m3s, init3, row3, col3, norm3)

    a_nodes, b_nodes = _tc_ab(agg3, b3r, m1a, m1b, c1r)
    z1 = _sc_edge(a_nodes, b_nodes, row3, col3, ew3, m1w)

    out = _tc_mlp(z1, M2, c2r, m3p, c3s)
    return out[:E, 0]


# speedup vs baseline: 4.7915x; 1.2943x over previous
import jax
import jax.numpy as jnp
from jax import lax
from jax.experimental import pallas as pl
from jax.experimental.pallas import tpu as pltpu
from jax.experimental.pallas import tpu_sc as plsc

N = 10000
E = 160000
D = 256

N_PAD = 10240
E_PAD = 163840
NC = 2
NS = 16
NW = NC * NS
EPW = E_PAD // NW
EPS = E_PAD // NS
CH = 128
CJ = EPS // CH
RPS = N_PAD // NS

_f32 = jnp.float32
_i32 = jnp.int32


def _mesh():
    return plsc.VectorSubcoreMesh(core_axis_name="c", subcore_axis_name="s")


def _sc_deg_body(col_hbm, ew_hbm, degp_hbm, colv, ewv, bins):
    c = lax.axis_index("c")
    s = lax.axis_index("s")
    wid = s * NC + c
    pltpu.sync_copy(col_hbm.at[wid], colv)
    pltpu.sync_copy(ew_hbm.at[wid], ewv)

    def zero_body(i, _):
        bins[pl.ds(i * 16, 16)] = jnp.zeros((16,), _f32)
        return 0

    lax.fori_loop(0, N_PAD // 16, zero_body, 0)

    def add_body(i, _):
        ci = colv[pl.ds(i * 16, 16)]
        wi = ewv[pl.ds(i * 16, 16)]
        plsc.addupdate_scatter(bins, [ci], wi)
        return 0

    lax.fori_loop(0, EPW // 16, add_body, 0)
    pltpu.sync_copy(bins, degp_hbm.at[wid])


def _sc_deg(col2, ew2):
    fn = pl.kernel(
        _sc_deg_body,
        out_type=jax.ShapeDtypeStruct((NW, N_PAD), _f32),
        mesh=_mesh(),
        compiler_params=pltpu.CompilerParams(needs_layout_passes=False, use_tc_tiling_on_sc=False),
        scratch_types=[
            pltpu.VMEM((EPW,), _i32),
            pltpu.VMEM((EPW,), _f32),
            pltpu.VMEM((N_PAD,), _f32),
        ],
    )
    return fn(col2, ew2)


def _scale_chunk(gbuf, normv, j, width):
    @plsc.parallel_loop(0, CH, unroll=4)
    def edge_body(e):
        ns = plsc.load_gather(
            normv, [jnp.full((16,), j, _i32), jnp.full((16,), e, _i32)]
        )
        for k in range(width // 16):
            sl = pl.ds(k * 16, 16)
            gbuf[e, sl] = gbuf[e, sl] * ns


def _agg_loop(m_hbm, init_hbm, out_hbm, rowv, colv, normv, gb, acc, sg, ss):
    c = lax.axis_index("c")
    s = lax.axis_index("s")
    for p in range(2):
        q = c * 2 + p
        pltpu.sync_copy(init_hbm.at[q, pl.ds(s * RPS, RPS)], acc.at[pl.ds(s * RPS, RPS)])
        plsc.subcore_barrier()

        pltpu.async_copy(m_hbm.at[q].at[rowv.at[0]], gb[0], sg[0])

        def step2(j2, _):
            for b in range(2):
                j = 2 * j2 + b
                jn = jnp.minimum(j + 1, CJ - 1)
                gX, sgX, ssX = gb[b], sg[b], ss[b]
                gY, sgY, ssY = gb[1 - b], sg[1 - b], ss[1 - b]
                if b == 0:
                    @pl.when(j2 > 0)
                    def _():
                        pltpu.make_async_copy(gY, acc.at[colv.at[jn]], ssY).wait()
                else:
                    pltpu.make_async_copy(gY, acc.at[colv.at[jn]], ssY).wait()
                pltpu.async_copy(m_hbm.at[q].at[rowv.at[jn]], gY, sgY)
                pltpu.make_async_copy(m_hbm.at[q].at[rowv.at[jn]], gX, sgX).wait()
                _scale_chunk(gX, normv, j, 64)
                pltpu.async_copy(gX, acc.at[colv.at[j]], ssX, add=True)
            return 0

        lax.fori_loop(0, CJ // 2, step2, 0)
        pltpu.make_async_copy(m_hbm.at[q].at[rowv.at[0]], gb[0], sg[0]).wait()
        pltpu.make_async_copy(gb[1], acc.at[colv.at[0]], ss[1]).wait()
        plsc.subcore_barrier()
        pltpu.sync_copy(acc.at[pl.ds(s * RPS, RPS)], out_hbm.at[q, pl.ds(s * RPS, RPS)])
        plsc.subcore_barrier()


def _sc_agg1_body(m_hbm, init_hbm, row_hbm, col_hbm, ew_hbm, dinv_hbm,
                  out_hbm, norm_hbm,
                  rowv, colv, ewv, normv, dinvv, gbuf0, gbuf1, acc,
                  sg0, sg1, ss0, ss1):
    c = lax.axis_index("c")
    s = lax.axis_index("s")
    pltpu.sync_copy(row_hbm.at[s], rowv)
    pltpu.sync_copy(col_hbm.at[s], colv)
    pltpu.sync_copy(ew_hbm.at[s], ewv)
    pltpu.sync_copy(dinv_hbm, dinvv)

    def norm_body(j, _):
        for k in range(8):
            sl = pl.ds(k * 16, 16)
            r16 = rowv[j, sl]
            c16 = colv[j, sl]
            w16 = ewv[j, sl]
            normv[j, sl] = (
                plsc.load_gather(dinvv, [r16]) * w16 * plsc.load_gather(dinvv, [c16])
            )
        return 0

    lax.fori_loop(0, CJ, norm_body, 0)

    @pl.when(c == 0)
    def _():
        pltpu.sync_copy(normv, norm_hbm.at[s])

    _agg_loop(m_hbm, init_hbm, out_hbm, rowv, colv, normv,
              (gbuf0, gbuf1), acc, (sg0, sg1), (ss0, ss1))


def _sc_agg2_body(m_hbm, init_hbm, row_hbm, col_hbm, norm_hbm, out_hbm,
                  rowv, colv, normv, gbuf0, gbuf1, acc, sg0, sg1, ss0, ss1):
    s = lax.axis_index("s")
    pltpu.sync_copy(row_hbm.at[s], rowv)
    pltpu.sync_copy(col_hbm.at[s], colv)
    pltpu.sync_copy(norm_hbm.at[s], normv)
    _agg_loop(m_hbm, init_hbm, out_hbm, rowv, colv, normv,
              (gbuf0, gbuf1), acc, (sg0, sg1), (ss0, ss1))


def _sc_agg1(m, init, row3, col3, ew3, dinv):
    fn = pl.kernel(
        _sc_agg1_body,
        out_type=(
            jax.ShapeDtypeStruct((4, N_PAD, 64), _f32),
            jax.ShapeDtypeStruct((NS, CJ, CH), _f32),
        ),
        mesh=_mesh(),
        compiler_params=pltpu.CompilerParams(needs_layout_passes=False, use_tc_tiling_on_sc=False),
        scratch_types=[
            pltpu.VMEM((CJ, CH), _i32),
            pltpu.VMEM((CJ, CH), _i32),
            pltpu.VMEM((CJ, CH), _f32),
            pltpu.VMEM((CJ, CH), _f32),
            pltpu.VMEM((N_PAD,), _f32),
            pltpu.VMEM((CH, 64), _f32),
            pltpu.VMEM((CH, 64), _f32),
            pltpu.VMEM_SHARED((N_PAD, 64), _f32),
            pltpu.SemaphoreType.DMA,
            pltpu.SemaphoreType.DMA,
            pltpu.SemaphoreType.DMA,
            pltpu.SemaphoreType.DMA,
        ],
    )
    return fn(m, init, row3, col3, ew3, dinv)


def _sc_agg2(m, init, row3, col3, norm3):
    fn = pl.kernel(
        _sc_agg2_body,
        out_type=jax.ShapeDtypeStruct((4, N_PAD, 64), _f32),
        mesh=_mesh(),
        compiler_params=pltpu.CompilerParams(needs_layout_passes=False, use_tc_tiling_on_sc=False),
        scratch_types=[
            pltpu.VMEM((CJ, CH), _i32),
            pltpu.VMEM((CJ, CH), _i32),
            pltpu.VMEM((CJ, CH), _f32),
            pltpu.VMEM((CH, 64), _f32),
            pltpu.VMEM((CH, 64), _f32),
            pltpu.VMEM_SHARED((N_PAD, 64), _f32),
            pltpu.SemaphoreType.DMA,
            pltpu.SemaphoreType.DMA,
            pltpu.SemaphoreType.DMA,
            pltpu.SemaphoreType.DMA,
        ],
    )
    return fn(m, init, row3, col3, norm3)


def _round_bf16(v):
    u = lax.bitcast_convert_type(v, _i32)
    lsb = lax.shift_right_logical(u, 16) & 1
    u = (u + 0x7FFF + lsb) & jnp.int32(-65536)
    return lax.bitcast_convert_type(u, _f32)


def _sc_edge_body(a_hbm, b_hbm, row_hbm, col_hbm, ew_hbm, m1w_hbm, z1_hbm,
                  rowv, colv, ewv, m1wv,
                  ab0, bb0, ab1, bb1,
                  sa0, sb0, sa1, sb1, w0, w1):
    c = lax.axis_index("c")
    s = lax.axis_index("s")
    pltpu.sync_copy(row_hbm.at[s], rowv)
    pltpu.sync_copy(col_hbm.at[s], colv)
    pltpu.sync_copy(ew_hbm.at[s], ewv)
    pltpu.sync_copy(m1w_hbm.at[c], m1wv)
    @plsc.parallel_loop(0, CJ, unroll=2)
    def _round_ew(i):
        for k in range(CH // 16):
            sl = pl.ds(k * 16, 16)
            ewv[i, sl] = _round_bf16(ewv[i, sl])

    w16 = [_round_bf16(m1wv[pl.ds(k * 16, 16)]) for k in range(8)]
    ab = (ab0, ab1)
    bb = (bb0, bb1)
    sa = (sa0, sa1)
    sb = (sb0, sb1)
    wr = (w0, w1)

    def combine(abufX, bbufX, j):
        @plsc.parallel_loop(0, CH, unroll=4)
        def edge_body(e):
            es = plsc.load_gather(
                ewv, [jnp.full((16,), j, _i32), jnp.full((16,), e, _i32)]
            )
            for k in range(8):
                sl = pl.ds(k * 16, 16)
                abufX[e, sl] = jnp.maximum(
                    abufX[e, sl] + bbufX[e, sl] + es * w16[k], 0.0
                )

    pltpu.async_copy(a_hbm.at[c].at[rowv.at[0]], ab[0], sa[0])
    pltpu.async_copy(b_hbm.at[c].at[colv.at[0]], bb[0], sb[0])

    def step2(j2, _):
        for b in range(2):
            j = 2 * j2 + b
            jn = jnp.minimum(j + 1, CJ - 1)
            abX, bbX, saX, sbX, wX = ab[b], bb[b], sa[b], sb[b], wr[b]
            abY, bbY, saY, sbY, wY = ab[1 - b], bb[1 - b], sa[1 - b], sb[1 - b], wr[1 - b]
            if b == 0:
                @pl.when(j2 > 0)
                def _():
                    pltpu.make_async_copy(
                        abY, z1_hbm.at[c, pl.ds(s * EPS, CH)], wY).wait()
            else:
                pltpu.make_async_copy(
                    abY, z1_hbm.at[c, pl.ds(s * EPS, CH)], wY).wait()
            pltpu.async_copy(a_hbm.at[c].at[rowv.at[jn]], abY, saY)
            pltpu.async_copy(b_hbm.at[c].at[colv.at[jn]], bbY, sbY)
            pltpu.make_async_copy(a_hbm.at[c].at[rowv.at[jn]], abX, saX).wait()
            pltpu.make_async_copy(b_hbm.at[c].at[colv.at[jn]], bbX, sbX).wait()
            combine(abX, bbX, j)
            pltpu.async_copy(abX, z1_hbm.at[c, pl.ds(s * EPS + j * CH, CH)], wX)
        return 0

    lax.fori_loop(0, CJ // 2, step2, 0)
    pltpu.make_async_copy(a_hbm.at[c].at[rowv.at[0]], ab[0], sa[0]).wait()
    pltpu.make_async_copy(b_hbm.at[c].at[colv.at[0]], bb[0], sb[0]).wait()
    pltpu.make_async_copy(ab[1], z1_hbm.at[c, pl.ds(s * EPS, CH)], wr[1]).wait()


def _sc_edge(a, b, row3, col3, ew3, m1w):
    fn = pl.kernel(
        _sc_edge_body,
        out_type=jax.ShapeDtypeStruct((NC, E_PAD, 128), _f32),
        mesh=_mesh(),
        compiler_params=pltpu.CompilerParams(needs_layout_passes=False, use_tc_tiling_on_sc=False),
        scratch_types=[
            pltpu.VMEM((CJ, CH), _i32),
            pltpu.VMEM((CJ, CH), _i32),
            pltpu.VMEM((CJ, CH), _f32),
            pltpu.VMEM((128,), _f32),
            pltpu.VMEM((CH, 128), _f32),
            pltpu.VMEM((CH, 128), _f32),
            pltpu.VMEM((CH, 128), _f32),
            pltpu.VMEM((CH, 128), _f32),
            pltpu.SemaphoreType.DMA,
            pltpu.SemaphoreType.DMA,
            pltpu.SemaphoreType.DMA,
            pltpu.SemaphoreType.DMA,
            pltpu.SemaphoreType.DMA,
            pltpu.SemaphoreType.DMA,
        ],
    )
    return fn(a, b, row3, col3, ew3, m1w)


BLKN = 1024
BLKE = 2048


def _split2(ref, val):
    ref[0] = val[:, :128]
    ref[1] = val[:, 128:]


def _split4(ref, val):
    for q in range(4):
        ref[q] = val[:, q * 64:(q + 1) * 64]


def _tc_prep_body(degp_ref, x_ref, w_ref, dinv_ref, m_ref, init_ref):
    deg = jnp.sum(degp_ref[...], axis=0) + 1.0
    dinv = 1.0 / jnp.sqrt(deg)
    dinv_ref[...] = dinv
    m = jnp.dot(x_ref[...], w_ref[...], preferred_element_type=_f32)
    _split4(m_ref, m)
    _split4(init_ref, m * (dinv * dinv)[:, None])


def _tc_prep(degp, x_p, w1):
    return pl.pallas_call(
        _tc_prep_body,
        grid=(N_PAD // BLKN,),
        in_specs=[
            pl.BlockSpec((NW, BLKN), lambda i: (0, i)),
            pl.BlockSpec((BLKN, D), lambda i: (i, 0)),
            pl.BlockSpec((D, D), lambda i: (0, 0)),
        ],
        out_specs=[
            pl.BlockSpec((BLKN,), lambda i: (i,)),
            pl.BlockSpec((4, BLKN, 64), lambda i: (0, i, 0)),
            pl.BlockSpec((4, BLKN, 64), lambda i: (0, i, 0)),
        ],
        out_shape=[
            jax.ShapeDtypeStruct((N_PAD,), _f32),
            jax.ShapeDtypeStruct((4, N_PAD, 64), _f32),
            jax.ShapeDtypeStruct((4, N_PAD, 64), _f32),
        ],
    )(degp, x_p, w1)


def _tc_layer_body(agg_ref, b_ref, w_ref, dinv_ref, m_ref, init_ref):
    h = jnp.concatenate([agg_ref[q] for q in range(4)], axis=-1) + b_ref[...]
    h = jnp.maximum(h, 0.0)
    dinv = dinv_ref[...]
    m = jnp.dot(h, w_ref[...], preferred_element_type=_f32)
    _split4(m_ref, m)
    _split4(init_ref, m * (dinv * dinv)[:, None])


def _tc_layer(agg, b_row, w, dinv):
    return pl.pallas_call(
        _tc_layer_body,
        grid=(N_PAD // BLKN,),
        in_specs=[
            pl.BlockSpec((4, BLKN, 64), lambda i: (0, i, 0)),
            pl.BlockSpec((1, D), lambda i: (0, 0)),
            pl.BlockSpec((D, D), lambda i: (0, 0)),
            pl.BlockSpec((BLKN,), lambda i: (i,)),
        ],
        out_specs=[
            pl.BlockSpec((4, BLKN, 64), lambda i: (0, i, 0)),
            pl.BlockSpec((4, BLKN, 64), lambda i: (0, i, 0)),
        ],
        out_shape=[
            jax.ShapeDtypeStruct((4, N_PAD, 64), _f32),
            jax.ShapeDtypeStruct((4, N_PAD, 64), _f32),
        ],
    )(agg, b_row, w, dinv)


def _tc_ab_body(agg_ref, b3_ref, m1a_ref, m1b_ref, c1_ref, a_ref, bo_ref):
    h = jnp.concatenate([agg_ref[q] for q in range(4)], axis=-1) + b3_ref[...]
    a = jnp.dot(h, m1a_ref[...], preferred_element_type=_f32) + c1_ref[...]
    bb = jnp.dot(h, m1b_ref[...], preferred_element_type=_f32)
    _split2(a_ref, a)
    _split2(bo_ref, bb)


def _tc_ab(agg, b3_row, m1a, m1b, c1_row):
    return pl.pallas_call(
        _tc_ab_body,
        grid=(N_PAD // BLKN,),
        in_specs=[
            pl.BlockSpec((4, BLKN, 64), lambda i: (0, i, 0)),
            pl.BlockSpec((1, D), lambda i: (0, 0)),
            pl.BlockSpec((D, D), lambda i: (0, 0)),
            pl.BlockSpec((D, D), lambda i: (0, 0)),
            pl.BlockSpec((1, D), lambda i: (0, 0)),
        ],
        out_specs=[
            pl.BlockSpec((NC, BLKN, 128), lambda i: (0, i, 0)),
            pl.BlockSpec((NC, BLKN, 128), lambda i: (0, i, 0)),
        ],
        out_shape=[
            jax.ShapeDtypeStruct((NC, N_PAD, 128), _f32),
            jax.ShapeDtypeStruct((NC, N_PAD, 128), _f32),
        ],
    )(agg, b3_row, m1a, m1b, c1_row)


def _tc_mlp_body(z1_ref, m2_ref, c2_ref, m3p_ref, c3_ref, out_ref):
    z = jnp.concatenate([z1_ref[0], z1_ref[1]], axis=-1)
    z2 = jnp.maximum(
        jnp.dot(z, m2_ref[...], preferred_element_type=_f32) + c2_ref[...], 0.0
    )
    t = jnp.dot(z2, m3p_ref[...], preferred_element_type=_f32)
    out_ref[...] = t[:, 0:1] + c3_ref[0, 0]


def _tc_mlp(z1, m2, c2_row, m3p, c3s):
    return pl.pallas_call(
        _tc_mlp_body,
        grid=(E_PAD // BLKE,),
        in_specs=[
            pl.BlockSpec((NC, BLKE, 128), lambda i: (0, i, 0)),
            pl.BlockSpec((D, D), lambda i: (0, 0)),
            pl.BlockSpec((1, D), lambda i: (0, 0)),
            pl.BlockSpec((D, 128), lambda i: (0, 0)),
            pl.BlockSpec(memory_space=pltpu.SMEM),
        ],
        out_specs=pl.BlockSpec((BLKE, 1), lambda i: (i, 0)),
        out_shape=jax.ShapeDtypeStruct((E_PAD, 1), _f32),
    )(z1, m2, c2_row, m3p, c3s)


def kernel(x, edge_index, edge_weight, W1, b1, W2, b2, W3, b3,
           M1, c1, M2, c2, M3, c3):
    row = edge_index[0]
    col = edge_index[1]
    x_p = jnp.pad(x, ((0, N_PAD - N), (0, 0)))
    row_p = jnp.pad(row, (0, E_PAD - E))
    col_p = jnp.pad(col, (0, E_PAD - E))
    ew_p = jnp.pad(edge_weight, (0, E_PAD - E))

    col2 = col_p.reshape(NW, EPW)
    ew2 = ew_p.reshape(NW, EPW)
    row3 = row_p.reshape(NS, CJ, CH)
    col3 = col_p.reshape(NS, CJ, CH)
    ew3 = ew_p.reshape(NS, CJ, CH)

    m1a = M1[:D]
    m1b = M1[D:2 * D]
    m1w = M1[2 * D].reshape(NC, 128)

    b1r = b1.reshape(1, D)
    b2r = b2.reshape(1, D)
    b3r = b3.reshape(1, D)
    c1r = c1.reshape(1, D)
    c2r = c2.reshape(1, D)
    c3s = c3.reshape(1, 1)
    m3p = jnp.pad(M3, ((0, 0), (0, 127)))

    degp = _sc_deg(col2, ew2)
    dinv, m1s, init1 = _tc_prep(degp, x_p, W1)

    agg1, norm3 = _sc_agg1(m1s, init1, row3, col3, ew3, dinv)
    m2s, init2 = _tc_layer(agg1, b1r, W2, dinv)
    agg2 = _sc_agg2(m2s, init2, row3, col3, norm3)
    m3s, init3 = _tc_layer(agg2, b2r, W3, dinv)
    agg3 = _sc_agg2(m3s, init3, row3, col3, norm3)

    a_nodes, b_nodes = _tc_ab(agg3, b3r, m1a, m1b, c1r)
    z1 = _sc_edge(a_nodes, b_nodes, row3, col3, ew3, m1w)

    out = _tc_mlp(z1, M2, c2r, m3p, c3s)
    return out[:E, 0]


# speedup vs baseline: 4.8642x; 1.0152x over previous
import jax
import jax.numpy as jnp
from jax import lax
from jax.experimental import pallas as pl
from jax.experimental.pallas import tpu as pltpu
from jax.experimental.pallas import tpu_sc as plsc

N = 10000
E = 160000
D = 256

N_PAD = 10240
E_PAD = 163840
NC = 2
NS = 16
NW = NC * NS
EPW = E_PAD // NW
EPS = E_PAD // NS
CH = 128
CJ = EPS // CH
RPS = N_PAD // NS

_f32 = jnp.float32
_i32 = jnp.int32


def _mesh():
    return plsc.VectorSubcoreMesh(core_axis_name="c", subcore_axis_name="s")


def _sc_deg_body(col_hbm, ew_hbm, degp_hbm, colv, ewv, bins):
    c = lax.axis_index("c")
    s = lax.axis_index("s")
    wid = s * NC + c
    pltpu.sync_copy(col_hbm.at[wid], colv)
    pltpu.sync_copy(ew_hbm.at[wid], ewv)

    def zero_body(i, _):
        bins[pl.ds(i * 16, 16)] = jnp.zeros((16,), _f32)
        return 0

    lax.fori_loop(0, N_PAD // 16, zero_body, 0)

    def add_body(i, _):
        ci = colv[pl.ds(i * 16, 16)]
        wi = ewv[pl.ds(i * 16, 16)]
        plsc.addupdate_scatter(bins, [ci], wi)
        return 0

    lax.fori_loop(0, EPW // 16, add_body, 0)
    pltpu.sync_copy(bins, degp_hbm.at[wid])


def _sc_deg(col2, ew2):
    fn = pl.kernel(
        _sc_deg_body,
        out_type=jax.ShapeDtypeStruct((NW, N_PAD), _f32),
        mesh=_mesh(),
        compiler_params=pltpu.CompilerParams(needs_layout_passes=False, use_tc_tiling_on_sc=False),
        scratch_types=[
            pltpu.VMEM((EPW,), _i32),
            pltpu.VMEM((EPW,), _f32),
            pltpu.VMEM((N_PAD,), _f32),
        ],
    )
    return fn(col2, ew2)


def _agg_loop(m_hbm, init_hbm, out_hbm, rowv, colv, normv, gb, sb, acc, sg, ss):
    c = lax.axis_index("c")
    s = lax.axis_index("s")
    for p in range(2):
        q = c * 2 + p
        pltpu.sync_copy(init_hbm.at[q, pl.ds(s * RPS, RPS)], acc.at[pl.ds(s * RPS, RPS)])
        plsc.subcore_barrier()

        pltpu.async_copy(m_hbm.at[q].at[rowv.at[0]], gb[0], sg[0])

        def step2(j2, _):
            for b in range(2):
                j = 2 * j2 + b
                jn = jnp.minimum(j + 1, CJ - 1)
                gX, sbX, sgX, ssX = gb[b], sb[b], sg[b], ss[b]
                gY, sgY = gb[1 - b], sg[1 - b]
                pltpu.async_copy(m_hbm.at[q].at[rowv.at[jn]], gY, sgY)
                pltpu.make_async_copy(m_hbm.at[q].at[rowv.at[jn]], gX, sgX).wait()
                if b == 0:
                    @pl.when(j2 > 0)
                    def _():
                        pltpu.make_async_copy(sbX, acc.at[colv.at[jn]], ssX).wait()
                else:
                    @pl.when(j2 > 0)
                    def _():
                        pltpu.make_async_copy(sbX, acc.at[colv.at[jn]], ssX).wait()

                @plsc.parallel_loop(0, CH, unroll=4)
                def edge_body(e):
                    ns = plsc.load_gather(
                        normv, [jnp.full((16,), j, _i32), jnp.full((16,), e, _i32)]
                    )
                    for k in range(4):
                        sl = pl.ds(k * 16, 16)
                        sbX[e, sl] = gX[e, sl] * ns

                pltpu.async_copy(sbX, acc.at[colv.at[j]], ssX, add=True)
            return 0

        lax.fori_loop(0, CJ // 2, step2, 0)
        pltpu.make_async_copy(m_hbm.at[q].at[rowv.at[0]], gb[0], sg[0]).wait()
        pltpu.make_async_copy(sb[0], acc.at[colv.at[0]], ss[0]).wait()
        pltpu.make_async_copy(sb[1], acc.at[colv.at[0]], ss[1]).wait()
        plsc.subcore_barrier()
        pltpu.sync_copy(acc.at[pl.ds(s * RPS, RPS)], out_hbm.at[q, pl.ds(s * RPS, RPS)])
        plsc.subcore_barrier()


def _sc_agg1_body(m_hbm, init_hbm, row_hbm, col_hbm, ew_hbm, dinv_hbm,
                  out_hbm, norm_hbm,
                  rowv, colv, ewv, normv, dinvv, gbuf0, gbuf1, sb0, sb1, acc,
                  sg0, sg1, ss0, ss1):
    c = lax.axis_index("c")
    s = lax.axis_index("s")
    pltpu.sync_copy(row_hbm.at[s], rowv)
    pltpu.sync_copy(col_hbm.at[s], colv)
    pltpu.sync_copy(ew_hbm.at[s], ewv)
    pltpu.sync_copy(dinv_hbm, dinvv)

    def norm_body(j, _):
        for k in range(8):
            sl = pl.ds(k * 16, 16)
            r16 = rowv[j, sl]
            c16 = colv[j, sl]
            w16 = ewv[j, sl]
            normv[j, sl] = (
                plsc.load_gather(dinvv, [r16]) * w16 * plsc.load_gather(dinvv, [c16])
            )
        return 0

    lax.fori_loop(0, CJ, norm_body, 0)

    @pl.when(c == 0)
    def _():
        pltpu.sync_copy(normv, norm_hbm.at[s])

    _agg_loop(m_hbm, init_hbm, out_hbm, rowv, colv, normv,
              (gbuf0, gbuf1), (sb0, sb1), acc, (sg0, sg1), (ss0, ss1))


def _sc_agg2_body(m_hbm, init_hbm, row_hbm, col_hbm, norm_hbm, out_hbm,
                  rowv, colv, normv, gbuf0, gbuf1, sb0, sb1, acc,
                  sg0, sg1, ss0, ss1):
    s = lax.axis_index("s")
    pltpu.sync_copy(row_hbm.at[s], rowv)
    pltpu.sync_copy(col_hbm.at[s], colv)
    pltpu.sync_copy(norm_hbm.at[s], normv)
    _agg_loop(m_hbm, init_hbm, out_hbm, rowv, colv, normv,
              (gbuf0, gbuf1), (sb0, sb1), acc, (sg0, sg1), (ss0, ss1))


def _sc_agg1(m, init, row3, col3, ew3, dinv):
    fn = pl.kernel(
        _sc_agg1_body,
        out_type=(
            jax.ShapeDtypeStruct((4, N_PAD, 64), _f32),
            jax.ShapeDtypeStruct((NS, CJ, CH), _f32),
        ),
        mesh=_mesh(),
        compiler_params=pltpu.CompilerParams(needs_layout_passes=False, use_tc_tiling_on_sc=False),
        scratch_types=[
            pltpu.VMEM((CJ, CH), _i32),
            pltpu.VMEM((CJ, CH), _i32),
            pltpu.VMEM((CJ, CH), _f32),
            pltpu.VMEM((CJ, CH), _f32),
            pltpu.VMEM((N_PAD,), _f32),
            pltpu.VMEM((CH, 64), _f32),
            pltpu.VMEM((CH, 64), _f32),
            pltpu.VMEM((CH, 64), _f32),
            pltpu.VMEM((CH, 64), _f32),
            pltpu.VMEM_SHARED((N_PAD, 64), _f32),
            pltpu.SemaphoreType.DMA,
            pltpu.SemaphoreType.DMA,
            pltpu.SemaphoreType.DMA,
            pltpu.SemaphoreType.DMA,
        ],
    )
    return fn(m, init, row3, col3, ew3, dinv)


def _sc_agg2(m, init, row3, col3, norm3):
    fn = pl.kernel(
        _sc_agg2_body,
        out_type=jax.ShapeDtypeStruct((4, N_PAD, 64), _f32),
        mesh=_mesh(),
        compiler_params=pltpu.CompilerParams(needs_layout_passes=False, use_tc_tiling_on_sc=False),
        scratch_types=[
            pltpu.VMEM((CJ, CH), _i32),
            pltpu.VMEM((CJ, CH), _i32),
            pltpu.VMEM((CJ, CH), _f32),
            pltpu.VMEM((CH, 64), _f32),
            pltpu.VMEM((CH, 64), _f32),
            pltpu.VMEM((CH, 64), _f32),
            pltpu.VMEM((CH, 64), _f32),
            pltpu.VMEM_SHARED((N_PAD, 64), _f32),
            pltpu.SemaphoreType.DMA,
            pltpu.SemaphoreType.DMA,
            pltpu.SemaphoreType.DMA,
            pltpu.SemaphoreType.DMA,
        ],
    )
    return fn(m, init, row3, col3, norm3)


def _round_bf16(v):
    u = lax.bitcast_convert_type(v, _i32)
    lsb = lax.shift_right_logical(u, 16) & 1
    u = (u + 0x7FFF + lsb) & jnp.int32(-65536)
    return lax.bitcast_convert_type(u, _f32)


def _sc_edge_body(a_hbm, b_hbm, row_hbm, col_hbm, ew_hbm, m1w_hbm, z1_hbm,
                  rowv, colv, ewv, m1wv,
                  ab0, bb0, ab1, bb1, zb0, zb1,
                  sa0, sb0, sa1, sb1, w0, w1):
    c = lax.axis_index("c")
    s = lax.axis_index("s")
    pltpu.sync_copy(row_hbm.at[s], rowv)
    pltpu.sync_copy(col_hbm.at[s], colv)
    pltpu.sync_copy(ew_hbm.at[s], ewv)
    pltpu.sync_copy(m1w_hbm.at[c], m1wv)
    @plsc.parallel_loop(0, CJ, unroll=2)
    def _round_ew(i):
        for k in range(CH // 16):
            sl = pl.ds(k * 16, 16)
            ewv[i, sl] = _round_bf16(ewv[i, sl])

    w16 = [_round_bf16(m1wv[pl.ds(k * 16, 16)]) for k in range(8)]
    ab = (ab0, ab1)
    bb = (bb0, bb1)
    zb = (zb0, zb1)
    sa = (sa0, sa1)
    sb = (sb0, sb1)
    wr = (w0, w1)

    pltpu.async_copy(a_hbm.at[c].at[rowv.at[0]], ab[0], sa[0])
    pltpu.async_copy(b_hbm.at[c].at[colv.at[0]], bb[0], sb[0])

    def step2(j2, _):
        for b in range(2):
            j = 2 * j2 + b
            jn = jnp.minimum(j + 1, CJ - 1)
            abX, bbX, zbX = ab[b], bb[b], zb[b]
            saX, sbX, wX = sa[b], sb[b], wr[b]
            abY, bbY, saY, sbY = ab[1 - b], bb[1 - b], sa[1 - b], sb[1 - b]
            pltpu.async_copy(a_hbm.at[c].at[rowv.at[jn]], abY, saY)
            pltpu.async_copy(b_hbm.at[c].at[colv.at[jn]], bbY, sbY)
            pltpu.make_async_copy(a_hbm.at[c].at[rowv.at[jn]], abX, saX).wait()
            pltpu.make_async_copy(b_hbm.at[c].at[colv.at[jn]], bbX, sbX).wait()
            @pl.when(j2 > 0)
            def _():
                pltpu.make_async_copy(
                    zbX, z1_hbm.at[c, pl.ds(s * EPS, CH)], wX).wait()

            @plsc.parallel_loop(0, CH, unroll=4)
            def edge_body(e):
                es = plsc.load_gather(
                    ewv, [jnp.full((16,), j, _i32), jnp.full((16,), e, _i32)]
                )
                for k in range(8):
                    sl = pl.ds(k * 16, 16)
                    zbX[e, sl] = jnp.maximum(
                        abX[e, sl] + bbX[e, sl] + es * w16[k], 0.0
                    )

            pltpu.async_copy(zbX, z1_hbm.at[c, pl.ds(s * EPS + j * CH, CH)], wX)
        return 0

    lax.fori_loop(0, CJ // 2, step2, 0)
    pltpu.make_async_copy(a_hbm.at[c].at[rowv.at[0]], ab[0], sa[0]).wait()
    pltpu.make_async_copy(b_hbm.at[c].at[colv.at[0]], bb[0], sb[0]).wait()
    pltpu.make_async_copy(zb[0], z1_hbm.at[c, pl.ds(s * EPS, CH)], wr[0]).wait()
    pltpu.make_async_copy(zb[1], z1_hbm.at[c, pl.ds(s * EPS, CH)], wr[1]).wait()


def _sc_edge(a, b, row3, col3, ew3, m1w):
    fn = pl.kernel(
        _sc_edge_body,
        out_type=jax.ShapeDtypeStruct((NC, E_PAD, 128), _f32),
        mesh=_mesh(),
        compiler_params=pltpu.CompilerParams(needs_layout_passes=False, use_tc_tiling_on_sc=False),
        scratch_types=[
            pltpu.VMEM((CJ, CH), _i32),
            pltpu.VMEM((CJ, CH), _i32),
            pltpu.VMEM((CJ, CH), _f32),
            pltpu.VMEM((128,), _f32),
            pltpu.VMEM((CH, 128), _f32),
            pltpu.VMEM((CH, 128), _f32),
            pltpu.VMEM((CH, 128), _f32),
            pltpu.VMEM((CH, 128), _f32),
            pltpu.VMEM((CH, 128), _f32),
            pltpu.VMEM((CH, 128), _f32),
            pltpu.SemaphoreType.DMA,
            pltpu.SemaphoreType.DMA,
            pltpu.SemaphoreType.DMA,
            pltpu.SemaphoreType.DMA,
            pltpu.SemaphoreType.DMA,
            pltpu.SemaphoreType.DMA,
        ],
    )
    return fn(a, b, row3, col3, ew3, m1w)


BLKN = 1024
BLKE = 2048


def _split2(ref, val):
    ref[0] = val[:, :128]
    ref[1] = val[:, 128:]


def _split4(ref, val):
    for q in range(4):
        ref[q] = val[:, q * 64:(q + 1) * 64]


def _tc_prep_body(degp_ref, x_ref, w_ref, dinv_ref, m_ref, init_ref):
    deg = jnp.sum(degp_ref[...], axis=0) + 1.0
    dinv = 1.0 / jnp.sqrt(deg)
    dinv_ref[...] = dinv
    m = jnp.dot(x_ref[...], w_ref[...], preferred_element_type=_f32)
    _split4(m_ref, m)
    _split4(init_ref, m * (dinv * dinv)[:, None])


def _tc_prep(degp, x_p, w1):
    return pl.pallas_call(
        _tc_prep_body,
        grid=(N_PAD // BLKN,),
        in_specs=[
            pl.BlockSpec((NW, BLKN), lambda i: (0, i)),
            pl.BlockSpec((BLKN, D), lambda i: (i, 0)),
            pl.BlockSpec((D, D), lambda i: (0, 0)),
        ],
        out_specs=[
            pl.BlockSpec((BLKN,), lambda i: (i,)),
            pl.BlockSpec((4, BLKN, 64), lambda i: (0, i, 0)),
            pl.BlockSpec((4, BLKN, 64), lambda i: (0, i, 0)),
        ],
        out_shape=[
            jax.ShapeDtypeStruct((N_PAD,), _f32),
            jax.ShapeDtypeStruct((4, N_PAD, 64), _f32),
            jax.ShapeDtypeStruct((4, N_PAD, 64), _f32),
        ],
    )(degp, x_p, w1)


def _tc_layer_body(agg_ref, b_ref, w_ref, dinv_ref, m_ref, init_ref):
    h = jnp.concatenate([agg_ref[q] for q in range(4)], axis=-1) + b_ref[...]
    h = jnp.maximum(h, 0.0)
    dinv = dinv_ref[...]
    m = jnp.dot(h, w_ref[...], preferred_element_type=_f32)
    _split4(m_ref, m)
    _split4(init_ref, m * (dinv * dinv)[:, None])


def _tc_layer(agg, b_row, w, dinv):
    return pl.pallas_call(
        _tc_layer_body,
        grid=(N_PAD // BLKN,),
        in_specs=[
            pl.BlockSpec((4, BLKN, 64), lambda i: (0, i, 0)),
            pl.BlockSpec((1, D), lambda i: (0, 0)),
            pl.BlockSpec((D, D), lambda i: (0, 0)),
            pl.BlockSpec((BLKN,), lambda i: (i,)),
        ],
        out_specs=[
            pl.BlockSpec((4, BLKN, 64), lambda i: (0, i, 0)),
            pl.BlockSpec((4, BLKN, 64), lambda i: (0, i, 0)),
        ],
        out_shape=[
            jax.ShapeDtypeStruct((4, N_PAD, 64), _f32),
            jax.ShapeDtypeStruct((4, N_PAD, 64), _f32),
        ],
    )(agg, b_row, w, dinv)


def _tc_ab_body(agg_ref, b3_ref, m1a_ref, m1b_ref, c1_ref, a_ref, bo_ref):
    h = jnp.concatenate([agg_ref[q] for q in range(4)], axis=-1) + b3_ref[...]
    a = jnp.dot(h, m1a_ref[...], preferred_element_type=_f32) + c1_ref[...]
    bb = jnp.dot(h, m1b_ref[...], preferred_element_type=_f32)
    _split2(a_ref, a)
    _split2(bo_ref, bb)


def _tc_ab(agg, b3_row, m1a, m1b, c1_row):
    return pl.pallas_call(
        _tc_ab_body,
        grid=(N_PAD // BLKN,),
        in_specs=[
            pl.BlockSpec((4, BLKN, 64), lambda i: (0, i, 0)),
            pl.BlockSpec((1, D), lambda i: (0, 0)),
            pl.BlockSpec((D, D), lambda i: (0, 0)),
            pl.BlockSpec((D, D), lambda i: (0, 0)),
            pl.BlockSpec((1, D), lambda i: (0, 0)),
        ],
        out_specs=[
            pl.BlockSpec((NC, BLKN, 128), lambda i: (0, i, 0)),
            pl.BlockSpec((NC, BLKN, 128), lambda i: (0, i, 0)),
        ],
        out_shape=[
            jax.ShapeDtypeStruct((NC, N_PAD, 128), _f32),
            jax.ShapeDtypeStruct((NC, N_PAD, 128), _f32),
        ],
    )(agg, b3_row, m1a, m1b, c1_row)


def _tc_mlp_body(z1_ref, m2_ref, c2_ref, m3p_ref, c3_ref, out_ref):
    z = jnp.concatenate([z1_ref[0], z1_ref[1]], axis=-1)
    z2 = jnp.maximum(
        jnp.dot(z, m2_ref[...], preferred_element_type=_f32) + c2_ref[...], 0.0
    )
    t = jnp.dot(z2, m3p_ref[...], preferred_element_type=_f32)
    out_ref[...] = t[:, 0:1] + c3_ref[0, 0]


def _tc_mlp(z1, m2, c2_row, m3p, c3s):
    return pl.pallas_call(
        _tc_mlp_body,
        grid=(E_PAD // BLKE,),
        in_specs=[
            pl.BlockSpec((NC, BLKE, 128), lambda i: (0, i, 0)),
            pl.BlockSpec((D, D), lambda i: (0, 0)),
            pl.BlockSpec((1, D), lambda i: (0, 0)),
            pl.BlockSpec((D, 128), lambda i: (0, 0)),
            pl.BlockSpec(memory_space=pltpu.SMEM),
        ],
        out_specs=pl.BlockSpec((BLKE, 1), lambda i: (i, 0)),
        out_shape=jax.ShapeDtypeStruct((E_PAD, 1), _f32),
    )(z1, m2, c2_row, m3p, c3s)


def kernel(x, edge_index, edge_weight, W1, b1, W2, b2, W3, b3,
           M1, c1, M2, c2, M3, c3):
    row = edge_index[0]
    col = edge_index[1]
    x_p = jnp.pad(x, ((0, N_PAD - N), (0, 0)))
    row_p = jnp.pad(row, (0, E_PAD - E))
    col_p = jnp.pad(col, (0, E_PAD - E))
    ew_p = jnp.pad(edge_weight, (0, E_PAD - E))

    col2 = col_p.reshape(NW, EPW)
    ew2 = ew_p.reshape(NW, EPW)
    row3 = row_p.reshape(NS, CJ, CH)
    col3 = col_p.reshape(NS, CJ, CH)
    ew3 = ew_p.reshape(NS, CJ, CH)

    m1a = M1[:D]
    m1b = M1[D:2 * D]
    m1w = M1[2 * D].reshape(NC, 128)

    b1r = b1.reshape(1, D)
    b2r = b2.reshape(1, D)
    b3r = b3.reshape(1, D)
    c1r = c1.reshape(1, D)
    c2r = c2.reshape(1, D)
    c3s = c3.reshape(1, 1)
    m3p = jnp.pad(M3, ((0, 0), (0, 127)))

    degp = _sc_deg(col2, ew2)
    dinv, m1s, init1 = _tc_prep(degp, x_p, W1)

    agg1, norm3 = _sc_agg1(m1s, init1, row3, col3, ew3, dinv)
    m2s, init2 = _tc_layer(agg1, b1r, W2, dinv)
    agg2 = _sc_agg2(m2s, init2, row3, col3, norm3)
    m3s, init3 = _tc_layer(agg2, b2r, W3, dinv)
    agg3 = _sc_agg2(m3s, init3, row3, col3, norm3)

    a_nodes, b_nodes = _tc_ab(agg3, b3r, m1a, m1b, c1r)
    z1 = _sc_edge(a_nodes, b_nodes, row3, col3, ew3, m1w)

    out = _tc_mlp(z1, M2, c2r, m3p, c3s)
    return out[:E, 0]
